# Initial kernel scaffold; baseline (speedup 1.0000x reference)
import jax
import jax.numpy as jnp
from jax.experimental import pallas as pl


def kernel(emb1, emb2, Wl1_0, Wr1_0, b1_0, Wl1_1, Wr1_1, b1_1, lin1_W, lin1_b, Wl2_0, Wr2_0, Wd2_0, b2_0, Wl2_1, Wr2_1, Wd2_1, b2_1, x1, edge_index1, batch1, x2_idx, edge_index2):
    raise NotImplementedError("write your pallas kernel here")


# speedup vs baseline: 4.7173x; 4.7173x over previous
import functools

import jax
import jax.numpy as jnp
from jax import lax
from jax.experimental import pallas as pl
from jax.experimental.pallas import tpu as pltpu
from jax.experimental.pallas import tpu_sc as plsc

D = 128
N1 = 10000
NP = 10240
E1 = 320000
G = 256
GP = 512
GM = 272
E2 = 4096

NC = 2
NS = 16
NW = NC * NS

EC = 80
EPW = E1 // NW
NCH = EPW // EC

RPW = NP // NW
RC = 64

_MESH = plsc.VectorSubcoreMesh(core_axis_name="c", subcore_axis_name="s")


def _wid():
    return lax.axis_index("s") * NC + lax.axis_index("c")


def _zero_rows(buf, nrows):
    z = jnp.zeros((16,), jnp.float32)

    @pl.loop(0, nrows)
    def _(r):
        for c in range(D // 16):
            buf[r, pl.ds(c * 16, 16)] = z


@functools.partial(
    pl.kernel,
    out_type=jax.ShapeDtypeStruct((NP, D), jnp.float32),
    mesh=_MESH,
    scratch_types=[
        pltpu.VMEM((RC,), jnp.int32),
        pltpu.VMEM((RC, D), jnp.float32),
        pltpu.SemaphoreType.DMA,
    ],
)
def _sc_gather(emb_hbm, idx_hbm, out_hbm, idx_v, rows_v, sem):
    base = _wid() * RPW

    @pl.loop(0, RPW // RC)
    def _(j):
        b = base + j * RC
        pltpu.sync_copy(idx_hbm.at[pl.ds(b, RC)], idx_v)
        pltpu.async_copy(emb_hbm.at[idx_v], rows_v, sem).wait()
        pltpu.sync_copy(rows_v, out_hbm.at[pl.ds(b, RC)])


@functools.partial(
    pl.kernel,
    out_type=(
        jax.ShapeDtypeStruct((NC, NP), jnp.float32),
        jax.ShapeDtypeStruct((NC, GP), jnp.float32),
        jax.ShapeDtypeStruct((NC, G * G), jnp.float32),
    ),
    mesh=_MESH,
    scratch_types=[
        pltpu.VMEM((EC,), jnp.int32),
        pltpu.VMEM((RC,), jnp.int32),
        pltpu.VMEM((RC,), jnp.int32),
        pltpu.VMEM((RC,), jnp.int32),
        pltpu.VMEM((RC,), jnp.int32),
        pltpu.VMEM((128,), jnp.float32),
        pltpu.VMEM((128,), jnp.float32),
        pltpu.VMEM_SHARED((NP,), jnp.float32),
        pltpu.VMEM_SHARED((GP,), jnp.float32),
        pltpu.VMEM_SHARED((G * G,), jnp.float32),
    ],
)
def _sc_counts(dst1_hbm, batch_hbm, src2_hbm, dst2_hbm,
               cnt_hbm, cntg_hbm, a2_hbm,
               didx, bidx, s2, d2, f2, ones_v, zeros_v,
               cnt_sh, cntg_sh, a2_sh):
    cid = lax.axis_index("c")
    sid = lax.axis_index("s")
    wid = sid * NC + cid

    one = jnp.ones((16,), jnp.float32)
    zero = jnp.zeros((16,), jnp.float32)
    for c in range(8):
        ones_v[pl.ds(c * 16, 16)] = one
        zeros_v[pl.ds(c * 16, 16)] = zero

    @pl.loop(0, NP // NS // 128)
    def _(k):
        pltpu.sync_copy(zeros_v, cnt_sh.at[pl.ds(sid * (NP // NS) + k * 128, 128)])

    pltpu.sync_copy(zeros_v.at[pl.ds(0, GP // NS)],
                    cntg_sh.at[pl.ds(sid * (GP // NS), GP // NS)])

    @pl.loop(0, G * G // NS // 128)
    def _(k):
        pltpu.sync_copy(zeros_v, a2_sh.at[pl.ds(sid * (G * G // NS) + k * 128, 128)])

    plsc.subcore_barrier()

    ebase = wid * EPW

    @pl.loop(0, NCH)
    def _(j):
        pltpu.sync_copy(dst1_hbm.at[pl.ds(ebase + j * EC, EC)], didx)
        pltpu.sync_copy(ones_v.at[pl.ds(0, EC)], cnt_sh.at[didx], add=True)

    rbase = wid * RPW

    @pl.loop(0, RPW // RC)
    def _(j):
        pltpu.sync_copy(batch_hbm.at[pl.ds(rbase + j * RC, RC)], bidx)
        pltpu.sync_copy(ones_v.at[pl.ds(0, RC)], cntg_sh.at[bidx], add=True)

    e2base = wid * (E2 // NW)

    @pl.loop(0, 2)
    def _(j):
        b = e2base + j * RC
        pltpu.sync_copy(src2_hbm.at[pl.ds(b, RC)], s2)
        pltpu.sync_copy(dst2_hbm.at[pl.ds(b, RC)], d2)
        for c in range(RC // 16):
            f2[pl.ds(c * 16, 16)] = (
                d2[pl.ds(c * 16, 16)] * G + s2[pl.ds(c * 16, 16)])
        pltpu.sync_copy(ones_v.at[pl.ds(0, RC)], a2_sh.at[f2], add=True)

    plsc.subcore_barrier()

    @pl.loop(0, NP // NS // 128)
    def _(k):
        o = sid * (NP // NS) + k * 128
        pltpu.sync_copy(cnt_sh.at[pl.ds(o, 128)], zeros_v)
        pltpu.sync_copy(zeros_v, cnt_hbm.at[cid, pl.ds(o, 128)])

    og = sid * (GP // NS)
    pltpu.sync_copy(cntg_sh.at[pl.ds(og, GP // NS)], zeros_v.at[pl.ds(0, GP // NS)])
    pltpu.sync_copy(zeros_v.at[pl.ds(0, GP // NS)],
                    cntg_hbm.at[cid, pl.ds(og, GP // NS)])

    @pl.loop(0, G * G // NS // 128)
    def _(k):
        o = sid * (G * G // NS) + k * 128
        pltpu.sync_copy(a2_sh.at[pl.ds(o, 128)], zeros_v)
        pltpu.sync_copy(zeros_v, a2_hbm.at[cid, pl.ds(o, 128)])


@functools.partial(
    pl.kernel,
    out_type=jax.ShapeDtypeStruct((NC, NP, D), jnp.float32),
    mesh=_MESH,
    scratch_types=[
        pltpu.VMEM((EC,), jnp.int32),
        pltpu.VMEM((EC,), jnp.int32),
        pltpu.VMEM((EC, D), jnp.float32),
        pltpu.VMEM_SHARED((NP, D), jnp.float32),
        pltpu.SemaphoreType.DMA,
    ],
)
def _sc_agg(x_hbm, src_hbm, dst_hbm, out_hbm, sidx, didx, rows, agg_sh, sem):
    cid = lax.axis_index("c")
    sid = lax.axis_index("s")
    wid = sid * NC + cid

    _zero_rows(rows, EC)

    @pl.loop(0, NP // NS // EC)
    def _(k):
        pltpu.sync_copy(rows, agg_sh.at[pl.ds(sid * (NP // NS) + k * EC, EC)])

    plsc.subcore_barrier()

    ebase = wid * EPW

    @pl.loop(0, NCH)
    def _(j):
        b = ebase + j * EC
        pltpu.sync_copy(src_hbm.at[pl.ds(b, EC)], sidx)
        pltpu.sync_copy(dst_hbm.at[pl.ds(b, EC)], didx)
        pltpu.async_copy(x_hbm.at[sidx], rows, sem).wait()
        pltpu.sync_copy(rows, agg_sh.at[didx], add=True)

    plsc.subcore_barrier()

    @pl.loop(0, NP // NS // EC)
    def _(k):
        o = sid * (NP // NS) + k * EC
        pltpu.sync_copy(agg_sh.at[pl.ds(o, EC)], rows)
        pltpu.sync_copy(rows, out_hbm.at[cid, pl.ds(o, EC)])


@functools.partial(
    pl.kernel,
    out_type=(
        jax.ShapeDtypeStruct((NC, GP, D), jnp.float32),
        jax.ShapeDtypeStruct((NW, GM, D), jnp.float32),
    ),
    mesh=_MESH,
    scratch_types=[
        pltpu.VMEM((RC,), jnp.int32),
        pltpu.VMEM((RC, D), jnp.float32),
        pltpu.VMEM((GM, D), jnp.float32),
        pltpu.VMEM_SHARED((GP, D), jnp.float32),
    ],
)
def _sc_pool(x_hbm, batch_hbm, sum_hbm, max_hbm, bidx, rows, lmax, sum_sh):
    cid = lax.axis_index("c")
    sid = lax.axis_index("s")
    wid = sid * NC + cid

    _zero_rows(lmax, GM)
    _zero_rows(rows, GP // NS)

    pltpu.sync_copy(rows.at[pl.ds(0, GP // NS)],
                    sum_sh.at[pl.ds(sid * (GP // NS), GP // NS)])
    plsc.subcore_barrier()

    base = wid * RPW

    @pl.loop(0, RPW // RC)
    def _(j):
        b = base + j * RC
        pltpu.sync_copy(batch_hbm.at[pl.ds(b, RC)], bidx)
        pltpu.sync_copy(x_hbm.at[pl.ds(b, RC)], rows)
        pltpu.sync_copy(rows, sum_sh.at[bidx], add=True)

        @pl.loop(0, RC // 16)
        def _(q):
            bvec = bidx[pl.ds(q * 16, 16)]
            for r in range(16):
                g = bvec[r]
                row = q * 16 + r
                for c in range(D // 16):
                    sl = pl.ds(c * 16, 16)
                    lmax[g, sl] = jnp.maximum(lmax[g, sl], rows[row, sl])

    pltpu.sync_copy(lmax, max_hbm.at[wid])

    plsc.subcore_barrier()
    o = sid * (GP // NS)
    pltpu.sync_copy(sum_sh.at[pl.ds(o, GP // NS)], rows.at[pl.ds(0, GP // NS)])
    pltpu.sync_copy(rows.at[pl.ds(0, GP // NS)],
                    sum_hbm.at[cid, pl.ds(o, GP // NS)])


_RB = 1024


def _tc_layer_body(aggp_ref, x_ref, cntp_ref, wl_ref, wr_ref, b_ref, o_ref):
    i = pl.program_id(0)
    cnt = cntp_ref[0, pl.ds(i * _RB, _RB)] + cntp_ref[1, pl.ds(i * _RB, _RB)]
    inv = 1.0 / jnp.maximum(cnt, 1.0)
    agg = (aggp_ref[0] + aggp_ref[1]) * inv.reshape(_RB, 1)
    acc = lax.dot_general(agg, wl_ref[...], (((1,), (1,)), ((), ())),
                          preferred_element_type=jnp.float32)
    acc += lax.dot_general(x_ref[...], wr_ref[...], (((1,), (1,)), ((), ())),
                           preferred_element_type=jnp.float32)
    o_ref[...] = jnp.maximum(acc + b_ref[...], 0.0)


def _tc_layer(aggp, x, cntp, wl, wr, b2d):
    return pl.pallas_call(
        _tc_layer_body,
        grid=(NP // _RB,),
        in_specs=[
            pl.BlockSpec((NC, _RB, D), lambda i: (0, i, 0)),
            pl.BlockSpec((_RB, D), lambda i: (i, 0)),
            pl.BlockSpec((NC, NP), lambda i: (0, 0)),
            pl.BlockSpec((D, D), lambda i: (0, 0)),
            pl.BlockSpec((D, D), lambda i: (0, 0)),
            pl.BlockSpec((1, D), lambda i: (0, 0)),
        ],
        out_specs=pl.BlockSpec((_RB, D), lambda i: (i, 0)),
        out_shape=jax.ShapeDtypeStruct((NP, D), jnp.float32),
    )(aggp, x, cntp, wl, wr, b2d)


def _tc_final_body(sump_ref, maxp_ref, cntgp_ref, a2p_ref,
                   lin1w_ref, lin1b_ref,
                   wl0_ref, wr0_ref, wd0_ref, b0_ref,
                   wl1_ref, wr1_ref, wd1_ref, b1_ref, o_ref):
    gm = maxp_ref[0, 0:G, :]
    for k in range(1, NW):
        gm = jnp.maximum(gm, maxp_ref[k, 0:G, :])
    ga = sump_ref[0, 0:G, :] + sump_ref[1, 0:G, :]
    cg = cntgp_ref[0, pl.ds(0, G)] + cntgp_ref[1, pl.ds(0, G)]
    ga = ga * (1.0 / jnp.maximum(cg, 1.0)).reshape(G, 1)
    xcat = jnp.concatenate([gm, ga], axis=1)
    xdrug = lax.dot_general(xcat, lin1w_ref[...], (((1,), (1,)), ((), ())),
                            preferred_element_type=jnp.float32)
    xdrug = jnp.maximum(xdrug + lin1b_ref[...], 0.0)

    a2 = a2p_ref[0] + a2p_ref[1]
    cnt2 = jnp.sum(a2, axis=1, keepdims=True)
    an = a2 / jnp.maximum(cnt2, 1.0)

    x2 = xdrug
    for (wl, wr, wd, b) in ((wl0_ref, wr0_ref, wd0_ref, b0_ref),
                            (wl1_ref, wr1_ref, wd1_ref, b1_ref)):
        agg2 = lax.dot_general(an, x2, (((1,), (0,)), ((), ())),
                               preferred_element_type=jnp.float32)
        acc = lax.dot_general(agg2, wl[...], (((1,), (1,)), ((), ())),
                              preferred_element_type=jnp.float32)
        acc += lax.dot_general(x2, wr[...], (((1,), (1,)), ((), ())),
                               preferred_element_type=jnp.float32)
        acc += lax.dot_general(xdrug, wd[...], (((1,), (1,)), ((), ())),
                               preferred_element_type=jnp.float32)
        x2 = jnp.maximum(acc + b[...], 0.0)
    o_ref[...] = x2


def _tc_final(sump, maxp, cntgp, a2p, lin1w, lin1b2d,
              wl0, wr0, wd0, b02d, wl1, wr1, wd1, b12d):
    return pl.pallas_call(
        _tc_final_body,
        out_shape=jax.ShapeDtypeStruct((G, D), jnp.float32),
    )(sump, maxp, cntgp, a2p, lin1w, lin1b2d,
      wl0, wr0, wd0, b02d, wl1, wr1, wd1, b12d)


def kernel(emb1, emb2, Wl1_0, Wr1_0, b1_0, Wl1_1, Wr1_1, b1_1, lin1_W, lin1_b,
           Wl2_0, Wr2_0, Wd2_0, b2_0, Wl2_1, Wr2_1, Wd2_1, b2_1,
           x1, edge_index1, batch1, x2_idx, edge_index2):
    x1p = jnp.concatenate([x1, jnp.full((NP - N1,), N1, jnp.int32)])
    batchp = jnp.concatenate([batch1, jnp.full((NP - N1,), G, jnp.int32)])
    src1, dst1 = edge_index1[0], edge_index1[1]
    src2, dst2 = edge_index2[0], edge_index2[1]

    xA = _sc_gather(emb1, x1p)
    cntp, cntgp, a2p_flat = _sc_counts(dst1, batchp, src2, dst2)
    a2p = a2p_flat.reshape(NC, G, G)

    b1_0_2d = b1_0.reshape(1, D)
    b1_1_2d = b1_1.reshape(1, D)

    aggp = _sc_agg(xA, src1, dst1)
    xA = _tc_layer(aggp, xA, cntp, Wl1_0, Wr1_0, b1_0_2d)
    aggp = _sc_agg(xA, src1, dst1)
    xA = _tc_layer(aggp, xA, cntp, Wl1_1, Wr1_1, b1_1_2d)

    sump, maxp = _sc_pool(xA, batchp)

    return _tc_final(sump, maxp, cntgp, a2p, lin1_W, lin1_b.reshape(1, D),
                     Wl2_0, Wr2_0, Wd2_0, b2_0.reshape(1, D),
                     Wl2_1, Wr2_1, Wd2_1, b2_1.reshape(1, D))


# speedup vs baseline: 10.4126x; 2.2073x over previous
import functools

import jax
import jax.numpy as jnp
from jax import lax
from jax.experimental import pallas as pl
from jax.experimental.pallas import tpu as pltpu
from jax.experimental.pallas import tpu_sc as plsc

D = 128
N1 = 10000
NP = 10240
E1 = 320000
G = 256
GP = 512
GM = 272
E2 = 4096

NC = 2
NS = 16
NW = NC * NS

EC = 80
ECH = E1 // EC
NCPW = ECH // NW
EPW = E1 // NW

RPW = NP // NW
RCH = RPW // EC
OB = NP // NS

_MESH = plsc.VectorSubcoreMesh(core_axis_name="c", subcore_axis_name="s")


def _wid():
    return lax.axis_index("s") * NC + lax.axis_index("c")


def _zero_rows(buf, nrows):
    z = jnp.zeros((16,), jnp.float32)

    @pl.loop(0, nrows)
    def _(r):
        for c in range(D // 16):
            buf[r, pl.ds(c * 16, 16)] = z


def _fill_1d(buf, n, val):
    v = jnp.full((16,), val, jnp.float32)

    @pl.loop(0, n // 16)
    def _(k):
        buf[pl.ds(k * 16, 16)] = v


@functools.partial(
    pl.kernel,
    out_type=(
        jax.ShapeDtypeStruct((NP, D), jnp.float32),
        jax.ShapeDtypeStruct((NC, NP), jnp.float32),
        jax.ShapeDtypeStruct((NC, GP), jnp.float32),
        jax.ShapeDtypeStruct((NC, G * G), jnp.float32),
    ),
    mesh=_MESH,
    scratch_types=[
        pltpu.VMEM((RPW,), jnp.int32),
        pltpu.VMEM((EPW,), jnp.int32),
        pltpu.VMEM((RPW,), jnp.int32),
        pltpu.VMEM((64,), jnp.int32),
        pltpu.VMEM((64,), jnp.int32),
        pltpu.VMEM((64,), jnp.int32),
        pltpu.VMEM((128,), jnp.float32),
        pltpu.VMEM((640,), jnp.float32),
        pltpu.VMEM((EC, D), jnp.float32),
        pltpu.VMEM((EC, D), jnp.float32),
        pltpu.VMEM_SHARED((NP,), jnp.float32),
        pltpu.VMEM_SHARED((GP,), jnp.float32),
        pltpu.VMEM_SHARED((G * G,), jnp.float32),
        pltpu.SemaphoreType.DMA,
        pltpu.SemaphoreType.DMA,
        pltpu.SemaphoreType.DMA,
    ],
)
def _sc_prep(emb_hbm, x1_hbm, dst1_hbm, batch_hbm, src2_hbm, dst2_hbm,
             xa_hbm, cnt_hbm, cntg_hbm, a2_hbm,
             gidx, didx, bidx, s2, d2, f2, ones_v, zb,
             rowsA, rowsB, cnt_sh, cntg_sh, a2_sh, semA, semB, ssem):
    cid = lax.axis_index("c")
    sid = lax.axis_index("s")
    wid = sid * NC + cid

    _fill_1d(ones_v, 128, 1.0)
    _fill_1d(zb, 640, 0.0)

    pltpu.sync_copy(zb, cnt_sh.at[pl.ds(sid * OB, OB)])
    pltpu.sync_copy(zb.at[pl.ds(0, GP // NS)],
                    cntg_sh.at[pl.ds(sid * (GP // NS), GP // NS)])
    a2pt = G * G // NS

    @pl.loop(0, 6)
    def _(k):
        pltpu.sync_copy(zb, a2_sh.at[pl.ds(sid * a2pt + k * 640, 640)])

    pltpu.sync_copy(zb.at[pl.ds(0, 256)],
                    a2_sh.at[pl.ds(sid * a2pt + 3840, 256)])

    rbase = wid * RPW
    pltpu.sync_copy(x1_hbm.at[pl.ds(rbase, RPW)], gidx)
    bufs = (rowsA, rowsB)
    sems = (semA, semB)
    pltpu.async_copy(emb_hbm.at[gidx.at[pl.ds(0, EC)]], rowsA, semA)
    for c in range(RCH):
        if c + 1 < RCH:
            pltpu.async_copy(emb_hbm.at[gidx.at[pl.ds((c + 1) * EC, EC)]],
                             bufs[(c + 1) % 2], sems[(c + 1) % 2])
        pltpu.make_async_copy(emb_hbm.at[pl.ds(0, EC)], bufs[c % 2],
                              sems[c % 2]).wait()
        pltpu.sync_copy(bufs[c % 2], xa_hbm.at[pl.ds(rbase + c * EC, EC)])

    plsc.subcore_barrier()

    pltpu.sync_copy(dst1_hbm.at[pl.ds(wid * EPW, EPW)], didx)

    @pl.loop(0, NCPW)
    def _(j):
        pltpu.sync_copy(ones_v.at[pl.ds(0, EC)],
                        cnt_sh.at[didx.at[pl.ds(j * EC, EC)]], add=True)

    pltpu.sync_copy(batch_hbm.at[pl.ds(wid * RPW, RPW)], bidx)
    for j in range(RCH):
        pltpu.sync_copy(ones_v.at[pl.ds(0, EC)],
                        cntg_sh.at[bidx.at[pl.ds(j * EC, EC)]], add=True)

    e2base = wid * (E2 // NW)
    for j in range(2):
        b = e2base + j * 64
        pltpu.sync_copy(src2_hbm.at[pl.ds(b, 64)], s2)
        pltpu.sync_copy(dst2_hbm.at[pl.ds(b, 64)], d2)
        for c in range(4):
            f2[pl.ds(c * 16, 16)] = (
                d2[pl.ds(c * 16, 16)] * G + s2[pl.ds(c * 16, 16)])
        pltpu.sync_copy(ones_v.at[pl.ds(0, 64)], a2_sh.at[f2], add=True)

    plsc.subcore_barrier()

    pltpu.sync_copy(cnt_sh.at[pl.ds(sid * OB, OB)], zb)
    pltpu.sync_copy(zb, cnt_hbm.at[cid, pl.ds(sid * OB, OB)])

    og = sid * (GP // NS)
    pltpu.sync_copy(cntg_sh.at[pl.ds(og, GP // NS)], zb.at[pl.ds(0, GP // NS)])
    pltpu.sync_copy(zb.at[pl.ds(0, GP // NS)],
                    cntg_hbm.at[cid, pl.ds(og, GP // NS)])

    @pl.loop(0, 6)
    def _(k):
        o = sid * a2pt + k * 640
        pltpu.sync_copy(a2_sh.at[pl.ds(o, 640)], zb)
        pltpu.sync_copy(zb, a2_hbm.at[cid, pl.ds(o, 640)])

    o = sid * a2pt + 3840
    pltpu.sync_copy(a2_sh.at[pl.ds(o, 256)], zb.at[pl.ds(0, 256)])
    pltpu.sync_copy(zb.at[pl.ds(0, 256)], a2_hbm.at[cid, pl.ds(o, 256)])


@functools.partial(
    pl.kernel,
    out_type=jax.ShapeDtypeStruct((NC, NP, D), jnp.float32),
    mesh=_MESH,
    scratch_types=[
        pltpu.VMEM((EPW,), jnp.int32),
        pltpu.VMEM((EPW,), jnp.int32),
        pltpu.VMEM((EC, D), jnp.float32),
        pltpu.VMEM((EC, D), jnp.float32),
        pltpu.VMEM_SHARED((NP, D), jnp.float32),
        pltpu.SemaphoreType.DMA,
        pltpu.SemaphoreType.DMA,
    ],
)
def _sc_agg(x_hbm, src_hbm, dst_hbm, out_hbm,
            sidx, didx, rowsA, rowsB, agg_sh, semA, semB):
    cid = lax.axis_index("c")
    sid = lax.axis_index("s")
    wid = sid * NC + cid

    _zero_rows(rowsA, EC)

    @pl.loop(0, OB // EC)
    def _(k):
        pltpu.sync_copy(rowsA, agg_sh.at[pl.ds(sid * OB + k * EC, EC)])

    pltpu.sync_copy(src_hbm.at[pl.ds(wid * EPW, EPW)], sidx)
    pltpu.sync_copy(dst_hbm.at[pl.ds(wid * EPW, EPW)], didx)
    plsc.subcore_barrier()

    def gath(c, buf, sem):
        pltpu.async_copy(x_hbm.at[sidx.at[pl.ds(c * EC, EC)]], buf, sem)

    def gwait(buf, sem):
        pltpu.make_async_copy(x_hbm.at[pl.ds(0, EC)], buf, sem).wait()

    def scat(c, buf):
        pltpu.sync_copy(buf, agg_sh.at[didx.at[pl.ds(c * EC, EC)]], add=True)

    gath(0, rowsA, semA)

    @pl.loop(0, (NCPW - 1) // 2)
    def _(j2):
        c = 2 * j2
        gath(c + 1, rowsB, semB)
        gwait(rowsA, semA)
        scat(c, rowsA)
        gath(c + 2, rowsA, semA)
        gwait(rowsB, semB)
        scat(c + 1, rowsB)

    gwait(rowsA, semA)
    scat(NCPW - 1, rowsA)

    plsc.subcore_barrier()

    @pl.loop(0, OB // EC)
    def _(k):
        o = sid * OB + k * EC
        pltpu.sync_copy(agg_sh.at[pl.ds(o, EC)], rowsA)
        pltpu.sync_copy(rowsA, out_hbm.at[cid, pl.ds(o, EC)])


@functools.partial(
    pl.kernel,
    out_type=(
        jax.ShapeDtypeStruct((NC, GP, D), jnp.float32),
        jax.ShapeDtypeStruct((NW, GM, D), jnp.float32),
    ),
    mesh=_MESH,
    scratch_types=[
        pltpu.VMEM((RPW,), jnp.int32),
        pltpu.VMEM((EC, D), jnp.float32),
        pltpu.VMEM((GM, D), jnp.float32),
        pltpu.VMEM_SHARED((GP, D), jnp.float32),
        pltpu.SemaphoreType.DMA,
    ],
)
def _sc_pool(x_hbm, batch_hbm, sum_hbm, max_hbm, bidx, rows, lmax, sum_sh, sem):
    cid = lax.axis_index("c")
    sid = lax.axis_index("s")
    wid = sid * NC + cid

    _zero_rows(lmax, GM)
    pltpu.sync_copy(lmax.at[pl.ds(0, GP // NS)],
                    sum_sh.at[pl.ds(sid * (GP // NS), GP // NS)])
    pltpu.sync_copy(batch_hbm.at[pl.ds(wid * RPW, RPW)], bidx)
    plsc.subcore_barrier()

    base = wid * RPW
    for j in range(RCH):
        pltpu.sync_copy(x_hbm.at[pl.ds(base + j * EC, EC)], rows)
        pltpu.sync_copy(rows, sum_sh.at[bidx.at[pl.ds(j * EC, EC)]], add=True)

        @pl.loop(0, EC // 16)
        def _(q):
            bvec = bidx[pl.ds(j * EC + q * 16, 16)]
            for r in range(16):
                g = bvec[r]
                row = q * 16 + r
                for c in range(D // 16):
                    sl = pl.ds(c * 16, 16)
                    lmax[g, sl] = jnp.maximum(lmax[g, sl], rows[row, sl])

    pltpu.sync_copy(lmax, max_hbm.at[wid])

    plsc.subcore_barrier()
    o = sid * (GP // NS)
    pltpu.sync_copy(sum_sh.at[pl.ds(o, GP // NS)], lmax.at[pl.ds(0, GP // NS)])
    pltpu.sync_copy(lmax.at[pl.ds(0, GP // NS)],
                    sum_hbm.at[cid, pl.ds(o, GP // NS)])


_RB = 1024


def _tc_layer_body(aggp_ref, x_ref, cntp_ref, wl_ref, wr_ref, b_ref, o_ref):
    i = pl.program_id(0)
    cnt = cntp_ref[0, pl.ds(i * _RB, _RB)] + cntp_ref[1, pl.ds(i * _RB, _RB)]
    inv = 1.0 / jnp.maximum(cnt, 1.0)
    agg = (aggp_ref[0] + aggp_ref[1]) * inv.reshape(_RB, 1)
    acc = lax.dot_general(agg, wl_ref[...], (((1,), (1,)), ((), ())),
                          preferred_element_type=jnp.float32)
    acc += lax.dot_general(x_ref[...], wr_ref[...], (((1,), (1,)), ((), ())),
                           preferred_element_type=jnp.float32)
    o_ref[...] = jnp.maximum(acc + b_ref[...], 0.0)


def _tc_layer(aggp, x, cntp, wl, wr, b2d):
    return pl.pallas_call(
        _tc_layer_body,
        grid=(NP // _RB,),
        in_specs=[
            pl.BlockSpec((NC, _RB, D), lambda i: (0, i, 0)),
            pl.BlockSpec((_RB, D), lambda i: (i, 0)),
            pl.BlockSpec((NC, NP), lambda i: (0, 0)),
            pl.BlockSpec((D, D), lambda i: (0, 0)),
            pl.BlockSpec((D, D), lambda i: (0, 0)),
            pl.BlockSpec((1, D), lambda i: (0, 0)),
        ],
        out_specs=pl.BlockSpec((_RB, D), lambda i: (i, 0)),
        out_shape=jax.ShapeDtypeStruct((NP, D), jnp.float32),
    )(aggp, x, cntp, wl, wr, b2d)


def _tc_final_body(sump_ref, maxp_ref, cntgp_ref, a2p_ref,
                   lin1w_ref, lin1b_ref,
                   wl0_ref, wr0_ref, wd0_ref, b0_ref,
                   wl1_ref, wr1_ref, wd1_ref, b1_ref, o_ref):
    gm = maxp_ref[0, 0:G, :]
    for k in range(1, NW):
        gm = jnp.maximum(gm, maxp_ref[k, 0:G, :])
    ga = sump_ref[0, 0:G, :] + sump_ref[1, 0:G, :]
    cg = cntgp_ref[0, pl.ds(0, G)] + cntgp_ref[1, pl.ds(0, G)]
    ga = ga * (1.0 / jnp.maximum(cg, 1.0)).reshape(G, 1)
    xcat = jnp.concatenate([gm, ga], axis=1)
    xdrug = lax.dot_general(xcat, lin1w_ref[...], (((1,), (1,)), ((), ())),
                            preferred_element_type=jnp.float32)
    xdrug = jnp.maximum(xdrug + lin1b_ref[...], 0.0)

    a2 = a2p_ref[0] + a2p_ref[1]
    cnt2 = jnp.sum(a2, axis=1, keepdims=True)
    an = a2 / jnp.maximum(cnt2, 1.0)

    x2 = xdrug
    for (wl, wr, wd, b) in ((wl0_ref, wr0_ref, wd0_ref, b0_ref),
                            (wl1_ref, wr1_ref, wd1_ref, b1_ref)):
        agg2 = lax.dot_general(an, x2, (((1,), (0,)), ((), ())),
                               preferred_element_type=jnp.float32)
        acc = lax.dot_general(agg2, wl[...], (((1,), (1,)), ((), ())),
                              preferred_element_type=jnp.float32)
        acc += lax.dot_general(x2, wr[...], (((1,), (1,)), ((), ())),
                               preferred_element_type=jnp.float32)
        acc += lax.dot_general(xdrug, wd[...], (((1,), (1,)), ((), ())),
                               preferred_element_type=jnp.float32)
        x2 = jnp.maximum(acc + b[...], 0.0)
    o_ref[...] = x2


def _tc_final(sump, maxp, cntgp, a2p, lin1w, lin1b2d,
              wl0, wr0, wd0, b02d, wl1, wr1, wd1, b12d):
    return pl.pallas_call(
        _tc_final_body,
        out_shape=jax.ShapeDtypeStruct((G, D), jnp.float32),
    )(sump, maxp, cntgp, a2p, lin1w, lin1b2d,
      wl0, wr0, wd0, b02d, wl1, wr1, wd1, b12d)


def kernel(emb1, emb2, Wl1_0, Wr1_0, b1_0, Wl1_1, Wr1_1, b1_1, lin1_W, lin1_b,
           Wl2_0, Wr2_0, Wd2_0, b2_0, Wl2_1, Wr2_1, Wd2_1, b2_1,
           x1, edge_index1, batch1, x2_idx, edge_index2):
    x1p = jnp.concatenate([x1, jnp.full((NP - N1,), N1, jnp.int32)])
    batchp = jnp.concatenate([batch1, jnp.full((NP - N1,), G, jnp.int32)])
    src1 = edge_index1[0]
    dst1 = edge_index1[1]
    src2, dst2 = edge_index2[0], edge_index2[1]

    xA, cntp, cntgp, a2p_flat = _sc_prep(emb1, x1p, dst1, batchp, src2, dst2)
    a2p = a2p_flat.reshape(NC, G, G)

    aggp = _sc_agg(xA, src1, dst1)
    xA = _tc_layer(aggp, xA, cntp, Wl1_0, Wr1_0, b1_0.reshape(1, D))
    aggp = _sc_agg(xA, src1, dst1)
    xA = _tc_layer(aggp, xA, cntp, Wl1_1, Wr1_1, b1_1.reshape(1, D))

    sump, maxp = _sc_pool(xA, batchp)

    return _tc_final(sump, maxp, cntgp, a2p, lin1_W, lin1_b.reshape(1, D),
                     Wl2_0, Wr2_0, Wd2_0, b2_0.reshape(1, D),
                     Wl2_1, Wr2_1, Wd2_1, b2_1.reshape(1, D))


# speedup vs baseline: 10.6512x; 1.0229x over previous
import functools

import jax
import jax.numpy as jnp
from jax import lax
from jax.experimental import pallas as pl
from jax.experimental.pallas import tpu as pltpu
from jax.experimental.pallas import tpu_sc as plsc

D = 128
N1 = 10000
NP = 10240
E1 = 320000
G = 256
GP = 512
GM = 272
E2 = 4096

NC = 2
NS = 16
NW = NC * NS

EC = 80
ECH = E1 // EC
NCPW = ECH // NW
EPW = E1 // NW

RPW = NP // NW
RCH = RPW // EC
OB = NP // NS

_MESH = plsc.VectorSubcoreMesh(core_axis_name="c", subcore_axis_name="s")


def _wid():
    return lax.axis_index("s") * NC + lax.axis_index("c")


def _zero_rows(buf, nrows):
    z = jnp.zeros((16,), jnp.float32)

    @pl.loop(0, nrows)
    def _(r):
        for c in range(D // 16):
            buf[r, pl.ds(c * 16, 16)] = z


def _fill_1d(buf, n, val):
    v = jnp.full((16,), val, jnp.float32)

    @pl.loop(0, n // 16)
    def _(k):
        buf[pl.ds(k * 16, 16)] = v


@functools.partial(
    pl.kernel,
    out_type=(
        jax.ShapeDtypeStruct((NP, D), jnp.float32),
        jax.ShapeDtypeStruct((NC, NP), jnp.float32),
        jax.ShapeDtypeStruct((NC, GP), jnp.float32),
        jax.ShapeDtypeStruct((NC, G * G), jnp.float32),
    ),
    mesh=_MESH,
    scratch_types=[
        pltpu.VMEM((RPW,), jnp.int32),
        pltpu.VMEM((EPW,), jnp.int32),
        pltpu.VMEM((RPW,), jnp.int32),
        pltpu.VMEM((64,), jnp.int32),
        pltpu.VMEM((64,), jnp.int32),
        pltpu.VMEM((64,), jnp.int32),
        pltpu.VMEM((128,), jnp.float32),
        pltpu.VMEM((640,), jnp.float32),
        pltpu.VMEM((EC, D), jnp.float32),
        pltpu.VMEM((EC, D), jnp.float32),
        pltpu.VMEM_SHARED((NP,), jnp.float32),
        pltpu.VMEM_SHARED((GP,), jnp.float32),
        pltpu.VMEM_SHARED((G * G,), jnp.float32),
        pltpu.SemaphoreType.DMA,
        pltpu.SemaphoreType.DMA,
        pltpu.SemaphoreType.DMA,
    ],
)
def _sc_prep(emb_hbm, x1_hbm, dst1_hbm, batch_hbm, src2_hbm, dst2_hbm,
             xa_hbm, cnt_hbm, cntg_hbm, a2_hbm,
             gidx, didx, bidx, s2, d2, f2, ones_v, zb,
             rowsA, rowsB, cnt_sh, cntg_sh, a2_sh, semA, semB, ssem):
    cid = lax.axis_index("c")
    sid = lax.axis_index("s")
    wid = sid * NC + cid

    _fill_1d(ones_v, 128, 1.0)
    _fill_1d(zb, 640, 0.0)

    pltpu.sync_copy(zb, cnt_sh.at[pl.ds(sid * OB, OB)])
    pltpu.sync_copy(zb.at[pl.ds(0, GP // NS)],
                    cntg_sh.at[pl.ds(sid * (GP // NS), GP // NS)])
    a2pt = G * G // NS

    @pl.loop(0, 6)
    def _(k):
        pltpu.sync_copy(zb, a2_sh.at[pl.ds(sid * a2pt + k * 640, 640)])

    pltpu.sync_copy(zb.at[pl.ds(0, 256)],
                    a2_sh.at[pl.ds(sid * a2pt + 3840, 256)])

    rbase = wid * RPW
    pltpu.sync_copy(x1_hbm.at[pl.ds(rbase, RPW)], gidx)
    bufs = (rowsA, rowsB)
    sems = (semA, semB)
    pltpu.async_copy(emb_hbm.at[gidx.at[pl.ds(0, EC)]], rowsA, semA)
    for c in range(RCH):
        if c + 1 < RCH:
            pltpu.async_copy(emb_hbm.at[gidx.at[pl.ds((c + 1) * EC, EC)]],
                             bufs[(c + 1) % 2], sems[(c + 1) % 2])
        pltpu.make_async_copy(emb_hbm.at[pl.ds(0, EC)], bufs[c % 2],
                              sems[c % 2]).wait()
        pltpu.sync_copy(bufs[c % 2], xa_hbm.at[pl.ds(rbase + c * EC, EC)])

    plsc.subcore_barrier()

    pltpu.sync_copy(dst1_hbm.at[pl.ds(wid * EPW, EPW)], didx)

    @pl.loop(0, 5)
    def _(blk):
        for j in range(25):
            pltpu.async_copy(
                ones_v.at[pl.ds(0, EC)],
                cnt_sh.at[didx.at[pl.ds((blk * 25 + j) * EC, EC)]],
                ssem, add=True)
        for j in range(25):
            pltpu.make_async_copy(
                ones_v.at[pl.ds(0, EC)],
                cnt_sh.at[didx.at[pl.ds((blk * 25 + j) * EC, EC)]],
                ssem).wait()

    pltpu.sync_copy(batch_hbm.at[pl.ds(wid * RPW, RPW)], bidx)
    for j in range(RCH):
        pltpu.sync_copy(ones_v.at[pl.ds(0, EC)],
                        cntg_sh.at[bidx.at[pl.ds(j * EC, EC)]], add=True)

    e2base = wid * (E2 // NW)
    for j in range(2):
        b = e2base + j * 64
        pltpu.sync_copy(src2_hbm.at[pl.ds(b, 64)], s2)
        pltpu.sync_copy(dst2_hbm.at[pl.ds(b, 64)], d2)
        for c in range(4):
            f2[pl.ds(c * 16, 16)] = (
                d2[pl.ds(c * 16, 16)] * G + s2[pl.ds(c * 16, 16)])
        pltpu.sync_copy(ones_v.at[pl.ds(0, 64)], a2_sh.at[f2], add=True)

    plsc.subcore_barrier()

    pltpu.sync_copy(cnt_sh.at[pl.ds(sid * OB, OB)], zb)
    pltpu.sync_copy(zb, cnt_hbm.at[cid, pl.ds(sid * OB, OB)])

    og = sid * (GP // NS)
    pltpu.sync_copy(cntg_sh.at[pl.ds(og, GP // NS)], zb.at[pl.ds(0, GP // NS)])
    pltpu.sync_copy(zb.at[pl.ds(0, GP // NS)],
                    cntg_hbm.at[cid, pl.ds(og, GP // NS)])

    @pl.loop(0, 6)
    def _(k):
        o = sid * a2pt + k * 640
        pltpu.sync_copy(a2_sh.at[pl.ds(o, 640)], zb)
        pltpu.sync_copy(zb, a2_hbm.at[cid, pl.ds(o, 640)])

    o = sid * a2pt + 3840
    pltpu.sync_copy(a2_sh.at[pl.ds(o, 256)], zb.at[pl.ds(0, 256)])
    pltpu.sync_copy(zb.at[pl.ds(0, 256)], a2_hbm.at[cid, pl.ds(o, 256)])


@functools.partial(
    pl.kernel,
    out_type=jax.ShapeDtypeStruct((NC, NP, D), jnp.float32),
    mesh=_MESH,
    scratch_types=[
        pltpu.VMEM((EPW,), jnp.int32),
        pltpu.VMEM((EPW,), jnp.int32),
        pltpu.VMEM((EC, D), jnp.float32),
        pltpu.VMEM((EC, D), jnp.float32),
        pltpu.VMEM_SHARED((NP, D), jnp.float32),
        pltpu.SemaphoreType.DMA,
        pltpu.SemaphoreType.DMA,
    ],
)
def _sc_agg(x_hbm, src_hbm, dst_hbm, out_hbm,
            sidx, didx, rowsA, rowsB, agg_sh, semA, semB):
    cid = lax.axis_index("c")
    sid = lax.axis_index("s")
    wid = sid * NC + cid

    _zero_rows(rowsA, EC)

    @pl.loop(0, OB // EC)
    def _(k):
        pltpu.sync_copy(rowsA, agg_sh.at[pl.ds(sid * OB + k * EC, EC)])

    pltpu.sync_copy(src_hbm.at[pl.ds(wid * EPW, EPW)], sidx)
    pltpu.sync_copy(dst_hbm.at[pl.ds(wid * EPW, EPW)], didx)
    plsc.subcore_barrier()

    def gath(c, buf, sem):
        pltpu.async_copy(x_hbm.at[sidx.at[pl.ds(c * EC, EC)]], buf, sem)

    def gwait(buf, sem):
        pltpu.make_async_copy(x_hbm.at[pl.ds(0, EC)], buf, sem).wait()

    def scat(c, buf):
        pltpu.sync_copy(buf, agg_sh.at[didx.at[pl.ds(c * EC, EC)]], add=True)

    gath(0, rowsA, semA)

    @pl.loop(0, (NCPW - 1) // 2)
    def _(j2):
        c = 2 * j2
        gath(c + 1, rowsB, semB)
        gwait(rowsA, semA)
        scat(c, rowsA)
        gath(c + 2, rowsA, semA)
        gwait(rowsB, semB)
        scat(c + 1, rowsB)

    gwait(rowsA, semA)
    scat(NCPW - 1, rowsA)

    plsc.subcore_barrier()

    @pl.loop(0, OB // EC)
    def _(k):
        o = sid * OB + k * EC
        pltpu.sync_copy(agg_sh.at[pl.ds(o, EC)], rowsA)
        pltpu.sync_copy(rowsA, out_hbm.at[cid, pl.ds(o, EC)])


@functools.partial(
    pl.kernel,
    out_type=(
        jax.ShapeDtypeStruct((NC, GP, D), jnp.float32),
        jax.ShapeDtypeStruct((NW, GM, D), jnp.float32),
    ),
    mesh=_MESH,
    scratch_types=[
        pltpu.VMEM((RPW,), jnp.int32),
        pltpu.VMEM((EC, D), jnp.float32),
        pltpu.VMEM((GM, D), jnp.float32),
        pltpu.VMEM_SHARED((GP, D), jnp.float32),
        pltpu.SemaphoreType.DMA,
    ],
)
def _sc_pool(x_hbm, batch_hbm, sum_hbm, max_hbm, bidx, rows, lmax, sum_sh, sem):
    cid = lax.axis_index("c")
    sid = lax.axis_index("s")
    wid = sid * NC + cid

    _zero_rows(lmax, GM)
    pltpu.sync_copy(lmax.at[pl.ds(0, GP // NS)],
                    sum_sh.at[pl.ds(sid * (GP // NS), GP // NS)])
    pltpu.sync_copy(batch_hbm.at[pl.ds(wid * RPW, RPW)], bidx)
    plsc.subcore_barrier()

    base = wid * RPW
    for j in range(RCH):
        pltpu.sync_copy(x_hbm.at[pl.ds(base + j * EC, EC)], rows)
        pltpu.sync_copy(rows, sum_sh.at[bidx.at[pl.ds(j * EC, EC)]], add=True)

        @pl.loop(0, EC // 16)
        def _(q):
            bvec = bidx[pl.ds(j * EC + q * 16, 16)]
            for r in range(16):
                g = bvec[r]
                row = q * 16 + r
                for c in range(D // 16):
                    sl = pl.ds(c * 16, 16)
                    lmax[g, sl] = jnp.maximum(lmax[g, sl], rows[row, sl])

    pltpu.sync_copy(lmax, max_hbm.at[wid])

    plsc.subcore_barrier()
    o = sid * (GP // NS)
    pltpu.sync_copy(sum_sh.at[pl.ds(o, GP // NS)], lmax.at[pl.ds(0, GP // NS)])
    pltpu.sync_copy(lmax.at[pl.ds(0, GP // NS)],
                    sum_hbm.at[cid, pl.ds(o, GP // NS)])


_RB = 1024


def _tc_layer_body(aggp_ref, x_ref, cntp_ref, wl_ref, wr_ref, b_ref, o_ref):
    i = pl.program_id(0)
    cnt = cntp_ref[0, pl.ds(i * _RB, _RB)] + cntp_ref[1, pl.ds(i * _RB, _RB)]
    inv = 1.0 / jnp.maximum(cnt, 1.0)
    agg = (aggp_ref[0] + aggp_ref[1]) * inv.reshape(_RB, 1)
    acc = lax.dot_general(agg, wl_ref[...], (((1,), (1,)), ((), ())),
                          preferred_element_type=jnp.float32)
    acc += lax.dot_general(x_ref[...], wr_ref[...], (((1,), (1,)), ((), ())),
                           preferred_element_type=jnp.float32)
    o_ref[...] = jnp.maximum(acc + b_ref[...], 0.0)


def _tc_layer(aggp, x, cntp, wl, wr, b2d):
    return pl.pallas_call(
        _tc_layer_body,
        grid=(NP // _RB,),
        in_specs=[
            pl.BlockSpec((NC, _RB, D), lambda i: (0, i, 0)),
            pl.BlockSpec((_RB, D), lambda i: (i, 0)),
            pl.BlockSpec((NC, NP), lambda i: (0, 0)),
            pl.BlockSpec((D, D), lambda i: (0, 0)),
            pl.BlockSpec((D, D), lambda i: (0, 0)),
            pl.BlockSpec((1, D), lambda i: (0, 0)),
        ],
        out_specs=pl.BlockSpec((_RB, D), lambda i: (i, 0)),
        out_shape=jax.ShapeDtypeStruct((NP, D), jnp.float32),
    )(aggp, x, cntp, wl, wr, b2d)


def _tc_final_body(sump_ref, maxp_ref, cntgp_ref, a2p_ref,
                   lin1w_ref, lin1b_ref,
                   wl0_ref, wr0_ref, wd0_ref, b0_ref,
                   wl1_ref, wr1_ref, wd1_ref, b1_ref, o_ref):
    gm = maxp_ref[0, 0:G, :]
    for k in range(1, NW):
        gm = jnp.maximum(gm, maxp_ref[k, 0:G, :])
    ga = sump_ref[0, 0:G, :] + sump_ref[1, 0:G, :]
    cg = cntgp_ref[0, pl.ds(0, G)] + cntgp_ref[1, pl.ds(0, G)]
    ga = ga * (1.0 / jnp.maximum(cg, 1.0)).reshape(G, 1)
    xcat = jnp.concatenate([gm, ga], axis=1)
    xdrug = lax.dot_general(xcat, lin1w_ref[...], (((1,), (1,)), ((), ())),
                            preferred_element_type=jnp.float32)
    xdrug = jnp.maximum(xdrug + lin1b_ref[...], 0.0)

    a2 = a2p_ref[0] + a2p_ref[1]
    cnt2 = jnp.sum(a2, axis=1, keepdims=True)
    an = a2 / jnp.maximum(cnt2, 1.0)

    x2 = xdrug
    for (wl, wr, wd, b) in ((wl0_ref, wr0_ref, wd0_ref, b0_ref),
                            (wl1_ref, wr1_ref, wd1_ref, b1_ref)):
        agg2 = lax.dot_general(an, x2, (((1,), (0,)), ((), ())),
                               preferred_element_type=jnp.float32)
        acc = lax.dot_general(agg2, wl[...], (((1,), (1,)), ((), ())),
                              preferred_element_type=jnp.float32)
        acc += lax.dot_general(x2, wr[...], (((1,), (1,)), ((), ())),
                               preferred_element_type=jnp.float32)
        acc += lax.dot_general(xdrug, wd[...], (((1,), (1,)), ((), ())),
                               preferred_element_type=jnp.float32)
        x2 = jnp.maximum(acc + b[...], 0.0)
    o_ref[...] = x2


def _tc_final(sump, maxp, cntgp, a2p, lin1w, lin1b2d,
              wl0, wr0, wd0, b02d, wl1, wr1, wd1, b12d):
    return pl.pallas_call(
        _tc_final_body,
        out_shape=jax.ShapeDtypeStruct((G, D), jnp.float32),
    )(sump, maxp, cntgp, a2p, lin1w, lin1b2d,
      wl0, wr0, wd0, b02d, wl1, wr1, wd1, b12d)


def kernel(emb1, emb2, Wl1_0, Wr1_0, b1_0, Wl1_1, Wr1_1, b1_1, lin1_W, lin1_b,
           Wl2_0, Wr2_0, Wd2_0, b2_0, Wl2_1, Wr2_1, Wd2_1, b2_1,
           x1, edge_index1, batch1, x2_idx, edge_index2):
    x1p = jnp.concatenate([x1, jnp.full((NP - N1,), N1, jnp.int32)])
    batchp = jnp.concatenate([batch1, jnp.full((NP - N1,), G, jnp.int32)])
    src1 = edge_index1[0]
    dst1 = edge_index1[1]
    src2, dst2 = edge_index2[0], edge_index2[1]

    xA, cntp, cntgp, a2p_flat = _sc_prep(emb1, x1p, dst1, batchp, src2, dst2)
    a2p = a2p_flat.reshape(NC, G, G)

    aggp = _sc_agg(xA, src1, dst1)
    xA = _tc_layer(aggp, xA, cntp, Wl1_0, Wr1_0, b1_0.reshape(1, D))
    aggp = _sc_agg(xA, src1, dst1)
    xA = _tc_layer(aggp, xA, cntp, Wl1_1, Wr1_1, b1_1.reshape(1, D))

    sump, maxp = _sc_pool(xA, batchp)

    return _tc_final(sump, maxp, cntgp, a2p, lin1_W, lin1_b.reshape(1, D),
                     Wl2_0, Wr2_0, Wd2_0, b2_0.reshape(1, D),
                     Wl2_1, Wr2_1, Wd2_1, b2_1.reshape(1, D))


# speedup vs baseline: 10.7779x; 1.0119x over previous
import functools

import jax
import jax.numpy as jnp
from jax import lax
from jax.experimental import pallas as pl
from jax.experimental.pallas import tpu as pltpu
from jax.experimental.pallas import tpu_sc as plsc

D = 128
N1 = 10000
NP = 10240
E1 = 320000
G = 256
GP = 512
GM = 272
E2 = 4096

NC = 2
NS = 16
NW = NC * NS

EC = 80
ECH = E1 // EC
NCPW = ECH // NW
EPW = E1 // NW

RPW = NP // NW
RCH = RPW // EC
OB = NP // NS

_MESH = plsc.VectorSubcoreMesh(core_axis_name="c", subcore_axis_name="s")


def _wid():
    return lax.axis_index("s") * NC + lax.axis_index("c")


def _zero_rows(buf, nrows):
    z = jnp.zeros((16,), jnp.float32)

    @pl.loop(0, nrows)
    def _(r):
        for c in range(D // 16):
            buf[r, pl.ds(c * 16, 16)] = z


def _fill_1d(buf, n, val):
    v = jnp.full((16,), val, jnp.float32)

    @pl.loop(0, n // 16)
    def _(k):
        buf[pl.ds(k * 16, 16)] = v


@functools.partial(
    pl.kernel,
    out_type=(
        jax.ShapeDtypeStruct((NP, D), jnp.float32),
        jax.ShapeDtypeStruct((NC, NP), jnp.float32),
        jax.ShapeDtypeStruct((NC, GP), jnp.float32),
        jax.ShapeDtypeStruct((NC, G * G), jnp.float32),
    ),
    mesh=_MESH,
    scratch_types=[
        pltpu.VMEM((RPW,), jnp.int32),
        pltpu.VMEM((EPW,), jnp.int32),
        pltpu.VMEM((RPW,), jnp.int32),
        pltpu.VMEM((64,), jnp.int32),
        pltpu.VMEM((64,), jnp.int32),
        pltpu.VMEM((64,), jnp.int32),
        pltpu.VMEM((128,), jnp.float32),
        pltpu.VMEM((640,), jnp.float32),
        pltpu.VMEM((EC, D), jnp.float32),
        pltpu.VMEM((EC, D), jnp.float32),
        pltpu.VMEM_SHARED((NP,), jnp.float32),
        pltpu.VMEM_SHARED((GP,), jnp.float32),
        pltpu.VMEM_SHARED((G * G,), jnp.float32),
        pltpu.SemaphoreType.DMA,
        pltpu.SemaphoreType.DMA,
        pltpu.SemaphoreType.DMA,
    ],
)
def _sc_prep(emb_hbm, x1_hbm, dst1_hbm, batch_hbm, src2_hbm, dst2_hbm,
             xa_hbm, cnt_hbm, cntg_hbm, a2_hbm,
             gidx, didx, bidx, s2, d2, f2, ones_v, zb,
             rowsA, rowsB, cnt_sh, cntg_sh, a2_sh, semA, semB, ssem):
    cid = lax.axis_index("c")
    sid = lax.axis_index("s")
    wid = sid * NC + cid

    _fill_1d(ones_v, 128, 1.0)
    _fill_1d(zb, 640, 0.0)

    pltpu.sync_copy(zb, cnt_sh.at[pl.ds(sid * OB, OB)])
    pltpu.sync_copy(zb.at[pl.ds(0, GP // NS)],
                    cntg_sh.at[pl.ds(sid * (GP // NS), GP // NS)])
    a2pt = G * G // NS

    @pl.loop(0, 6)
    def _(k):
        pltpu.sync_copy(zb, a2_sh.at[pl.ds(sid * a2pt + k * 640, 640)])

    pltpu.sync_copy(zb.at[pl.ds(0, 256)],
                    a2_sh.at[pl.ds(sid * a2pt + 3840, 256)])

    rbase = wid * RPW
    pltpu.sync_copy(x1_hbm.at[pl.ds(rbase, RPW)], gidx)
    bufs = (rowsA, rowsB)
    sems = (semA, semB)
    pltpu.async_copy(emb_hbm.at[gidx.at[pl.ds(0, EC)]], rowsA, semA)
    for c in range(RCH):
        if c + 1 < RCH:
            pltpu.async_copy(emb_hbm.at[gidx.at[pl.ds((c + 1) * EC, EC)]],
                             bufs[(c + 1) % 2], sems[(c + 1) % 2])
        pltpu.make_async_copy(emb_hbm.at[pl.ds(0, EC)], bufs[c % 2],
                              sems[c % 2]).wait()
        pltpu.sync_copy(bufs[c % 2], xa_hbm.at[pl.ds(rbase + c * EC, EC)])

    plsc.subcore_barrier()

    pltpu.sync_copy(dst1_hbm.at[pl.ds(wid * EPW, EPW)], didx)

    @pl.loop(0, 5)
    def _(blk):
        for j in range(25):
            pltpu.async_copy(
                ones_v.at[pl.ds(0, EC)],
                cnt_sh.at[didx.at[pl.ds((blk * 25 + j) * EC, EC)]],
                ssem, add=True)
        for j in range(25):
            pltpu.make_async_copy(
                ones_v.at[pl.ds(0, EC)],
                cnt_sh.at[didx.at[pl.ds((blk * 25 + j) * EC, EC)]],
                ssem).wait()

    pltpu.sync_copy(batch_hbm.at[pl.ds(wid * RPW, RPW)], bidx)
    for j in range(RCH):
        pltpu.sync_copy(ones_v.at[pl.ds(0, EC)],
                        cntg_sh.at[bidx.at[pl.ds(j * EC, EC)]], add=True)

    e2base = wid * (E2 // NW)
    for j in range(2):
        b = e2base + j * 64
        pltpu.sync_copy(src2_hbm.at[pl.ds(b, 64)], s2)
        pltpu.sync_copy(dst2_hbm.at[pl.ds(b, 64)], d2)
        for c in range(4):
            f2[pl.ds(c * 16, 16)] = (
                d2[pl.ds(c * 16, 16)] * G + s2[pl.ds(c * 16, 16)])
        pltpu.sync_copy(ones_v.at[pl.ds(0, 64)], a2_sh.at[f2], add=True)

    plsc.subcore_barrier()

    pltpu.sync_copy(cnt_sh.at[pl.ds(sid * OB, OB)], zb)
    pltpu.sync_copy(zb, cnt_hbm.at[cid, pl.ds(sid * OB, OB)])

    og = sid * (GP // NS)
    pltpu.sync_copy(cntg_sh.at[pl.ds(og, GP // NS)], zb.at[pl.ds(0, GP // NS)])
    pltpu.sync_copy(zb.at[pl.ds(0, GP // NS)],
                    cntg_hbm.at[cid, pl.ds(og, GP // NS)])

    @pl.loop(0, 6)
    def _(k):
        o = sid * a2pt + k * 640
        pltpu.sync_copy(a2_sh.at[pl.ds(o, 640)], zb)
        pltpu.sync_copy(zb, a2_hbm.at[cid, pl.ds(o, 640)])

    o = sid * a2pt + 3840
    pltpu.sync_copy(a2_sh.at[pl.ds(o, 256)], zb.at[pl.ds(0, 256)])
    pltpu.sync_copy(zb.at[pl.ds(0, 256)], a2_hbm.at[cid, pl.ds(o, 256)])


@functools.partial(
    pl.kernel,
    out_type=jax.ShapeDtypeStruct((NC, NP, D), jnp.float32),
    mesh=_MESH,
    scratch_types=[
        pltpu.VMEM((EPW,), jnp.int32),
        pltpu.VMEM((EPW,), jnp.int32),
        pltpu.VMEM((EC, D), jnp.float32),
        pltpu.VMEM((EC, D), jnp.float32),
        pltpu.VMEM_SHARED((NP, D), jnp.float32),
        pltpu.SemaphoreType.DMA,
        pltpu.SemaphoreType.DMA,
    ],
)
def _sc_agg(x_hbm, src_hbm, dst_hbm, out_hbm,
            sidx, didx, rowsA, rowsB, agg_sh, semA, semB):
    cid = lax.axis_index("c")
    sid = lax.axis_index("s")
    wid = sid * NC + cid

    _zero_rows(rowsA, EC)

    @pl.loop(0, OB // EC)
    def _(k):
        pltpu.sync_copy(rowsA, agg_sh.at[pl.ds(sid * OB + k * EC, EC)])

    pltpu.sync_copy(src_hbm.at[pl.ds(wid * EPW, EPW)], sidx)
    pltpu.sync_copy(dst_hbm.at[pl.ds(wid * EPW, EPW)], didx)
    plsc.subcore_barrier()

    def gath(c, buf, sem):
        pltpu.async_copy(x_hbm.at[sidx.at[pl.ds(c * EC, EC)]], buf, sem)

    def gwait(buf, sem):
        pltpu.make_async_copy(x_hbm.at[pl.ds(0, EC)], buf, sem).wait()

    def scat(c, buf):
        pltpu.sync_copy(buf, agg_sh.at[didx.at[pl.ds(c * EC, EC)]], add=True)

    gath(0, rowsA, semA)

    @pl.loop(0, (NCPW - 1) // 2)
    def _(j2):
        c = 2 * j2
        gath(c + 1, rowsB, semB)
        gwait(rowsA, semA)
        scat(c, rowsA)
        gath(c + 2, rowsA, semA)
        gwait(rowsB, semB)
        scat(c + 1, rowsB)

    gwait(rowsA, semA)
    scat(NCPW - 1, rowsA)

    plsc.subcore_barrier()

    obufs = (rowsA, rowsB)
    osems = (semA, semB)
    for k in range(OB // EC):
        o = sid * OB + k * EC
        b, sm = obufs[k % 2], osems[k % 2]
        if k >= 2:
            op = sid * OB + (k - 2) * EC
            pltpu.make_async_copy(b, out_hbm.at[cid, pl.ds(op, EC)], sm).wait()
        pltpu.sync_copy(agg_sh.at[pl.ds(o, EC)], b)
        pltpu.async_copy(b, out_hbm.at[cid, pl.ds(o, EC)], sm)
    for k in (OB // EC - 2, OB // EC - 1):
        o = sid * OB + k * EC
        pltpu.make_async_copy(obufs[k % 2], out_hbm.at[cid, pl.ds(o, EC)],
                              osems[k % 2]).wait()


@functools.partial(
    pl.kernel,
    out_type=(
        jax.ShapeDtypeStruct((NC, GP, D), jnp.float32),
        jax.ShapeDtypeStruct((NW, GM, D), jnp.float32),
    ),
    mesh=_MESH,
    scratch_types=[
        pltpu.VMEM((RPW,), jnp.int32),
        pltpu.VMEM((EC, D), jnp.float32),
        pltpu.VMEM((GM, D), jnp.float32),
        pltpu.VMEM_SHARED((GP, D), jnp.float32),
        pltpu.SemaphoreType.DMA,
    ],
)
def _sc_pool(x_hbm, batch_hbm, sum_hbm, max_hbm, bidx, rows, lmax, sum_sh, sem):
    cid = lax.axis_index("c")
    sid = lax.axis_index("s")
    wid = sid * NC + cid

    _zero_rows(lmax, GM)
    pltpu.sync_copy(lmax.at[pl.ds(0, GP // NS)],
                    sum_sh.at[pl.ds(sid * (GP // NS), GP // NS)])
    pltpu.sync_copy(batch_hbm.at[pl.ds(wid * RPW, RPW)], bidx)
    plsc.subcore_barrier()

    base = wid * RPW
    for j in range(RCH):
        pltpu.sync_copy(x_hbm.at[pl.ds(base + j * EC, EC)], rows)
        pltpu.sync_copy(rows, sum_sh.at[bidx.at[pl.ds(j * EC, EC)]], add=True)

        @pl.loop(0, EC // 16)
        def _(q):
            bvec = bidx[pl.ds(j * EC + q * 16, 16)]
            for r in range(16):
                g = bvec[r]
                row = q * 16 + r
                for c in range(D // 16):
                    sl = pl.ds(c * 16, 16)
                    lmax[g, sl] = jnp.maximum(lmax[g, sl], rows[row, sl])

    pltpu.sync_copy(lmax, max_hbm.at[wid])

    plsc.subcore_barrier()
    o = sid * (GP // NS)
    pltpu.sync_copy(sum_sh.at[pl.ds(o, GP // NS)], lmax.at[pl.ds(0, GP // NS)])
    pltpu.sync_copy(lmax.at[pl.ds(0, GP // NS)],
                    sum_hbm.at[cid, pl.ds(o, GP // NS)])


_RB = 1024


def _tc_layer_body(aggp_ref, x_ref, cntp_ref, wl_ref, wr_ref, b_ref, o_ref):
    i = pl.program_id(0)
    cnt = cntp_ref[0, pl.ds(i * _RB, _RB)] + cntp_ref[1, pl.ds(i * _RB, _RB)]
    inv = 1.0 / jnp.maximum(cnt, 1.0)
    agg = (aggp_ref[0] + aggp_ref[1]) * inv.reshape(_RB, 1)
    acc = lax.dot_general(agg, wl_ref[...], (((1,), (1,)), ((), ())),
                          preferred_element_type=jnp.float32)
    acc += lax.dot_general(x_ref[...], wr_ref[...], (((1,), (1,)), ((), ())),
                           preferred_element_type=jnp.float32)
    o_ref[...] = jnp.maximum(acc + b_ref[...], 0.0)


def _tc_layer(aggp, x, cntp, wl, wr, b2d):
    return pl.pallas_call(
        _tc_layer_body,
        grid=(NP // _RB,),
        in_specs=[
            pl.BlockSpec((NC, _RB, D), lambda i: (0, i, 0)),
            pl.BlockSpec((_RB, D), lambda i: (i, 0)),
            pl.BlockSpec((NC, NP), lambda i: (0, 0)),
            pl.BlockSpec((D, D), lambda i: (0, 0)),
            pl.BlockSpec((D, D), lambda i: (0, 0)),
            pl.BlockSpec((1, D), lambda i: (0, 0)),
        ],
        out_specs=pl.BlockSpec((_RB, D), lambda i: (i, 0)),
        out_shape=jax.ShapeDtypeStruct((NP, D), jnp.float32),
    )(aggp, x, cntp, wl, wr, b2d)


def _tc_final_body(sump_ref, maxp_ref, cntgp_ref, a2p_ref,
                   lin1w_ref, lin1b_ref,
                   wl0_ref, wr0_ref, wd0_ref, b0_ref,
                   wl1_ref, wr1_ref, wd1_ref, b1_ref, o_ref):
    gm = maxp_ref[0, 0:G, :]
    for k in range(1, NW):
        gm = jnp.maximum(gm, maxp_ref[k, 0:G, :])
    ga = sump_ref[0, 0:G, :] + sump_ref[1, 0:G, :]
    cg = cntgp_ref[0, pl.ds(0, G)] + cntgp_ref[1, pl.ds(0, G)]
    ga = ga * (1.0 / jnp.maximum(cg, 1.0)).reshape(G, 1)
    xcat = jnp.concatenate([gm, ga], axis=1)
    xdrug = lax.dot_general(xcat, lin1w_ref[...], (((1,), (1,)), ((), ())),
                            preferred_element_type=jnp.float32)
    xdrug = jnp.maximum(xdrug + lin1b_ref[...], 0.0)

    a2 = a2p_ref[0] + a2p_ref[1]
    cnt2 = jnp.sum(a2, axis=1, keepdims=True)
    an = a2 / jnp.maximum(cnt2, 1.0)

    x2 = xdrug
    for (wl, wr, wd, b) in ((wl0_ref, wr0_ref, wd0_ref, b0_ref),
                            (wl1_ref, wr1_ref, wd1_ref, b1_ref)):
        agg2 = lax.dot_general(an, x2, (((1,), (0,)), ((), ())),
                               preferred_element_type=jnp.float32)
        acc = lax.dot_general(agg2, wl[...], (((1,), (1,)), ((), ())),
                              preferred_element_type=jnp.float32)
        acc += lax.dot_general(x2, wr[...], (((1,), (1,)), ((), ())),
                               preferred_element_type=jnp.float32)
        acc += lax.dot_general(xdrug, wd[...], (((1,), (1,)), ((), ())),
                               preferred_element_type=jnp.float32)
        x2 = jnp.maximum(acc + b[...], 0.0)
    o_ref[...] = x2


def _tc_final(sump, maxp, cntgp, a2p, lin1w, lin1b2d,
              wl0, wr0, wd0, b02d, wl1, wr1, wd1, b12d):
    return pl.pallas_call(
        _tc_final_body,
        out_shape=jax.ShapeDtypeStruct((G, D), jnp.float32),
    )(sump, maxp, cntgp, a2p, lin1w, lin1b2d,
      wl0, wr0, wd0, b02d, wl1, wr1, wd1, b12d)


def kernel(emb1, emb2, Wl1_0, Wr1_0, b1_0, Wl1_1, Wr1_1, b1_1, lin1_W, lin1_b,
           Wl2_0, Wr2_0, Wd2_0, b2_0, Wl2_1, Wr2_1, Wd2_1, b2_1,
           x1, edge_index1, batch1, x2_idx, edge_index2):
    x1p = jnp.concatenate([x1, jnp.full((NP - N1,), N1, jnp.int32)])
    batchp = jnp.concatenate([batch1, jnp.full((NP - N1,), G, jnp.int32)])
    src1 = edge_index1[0]
    dst1 = edge_index1[1]
    src2, dst2 = edge_index2[0], edge_index2[1]

    xA, cntp, cntgp, a2p_flat = _sc_prep(emb1, x1p, dst1, batchp, src2, dst2)
    a2p = a2p_flat.reshape(NC, G, G)

    aggp = _sc_agg(xA, src1, dst1)
    xA = _tc_layer(aggp, xA, cntp, Wl1_0, Wr1_0, b1_0.reshape(1, D))
    aggp = _sc_agg(xA, src1, dst1)
    xA = _tc_layer(aggp, xA, cntp, Wl1_1, Wr1_1, b1_1.reshape(1, D))

    sump, maxp = _sc_pool(xA, batchp)

    return _tc_final(sump, maxp, cntgp, a2p, lin1_W, lin1_b.reshape(1, D),
                     Wl2_0, Wr2_0, Wd2_0, b2_0.reshape(1, D),
                     Wl2_1, Wr2_1, Wd2_1, b2_1.reshape(1, D))


# speedup vs baseline: 10.8973x; 1.0111x over previous
import functools

import jax
import jax.numpy as jnp
from jax import lax
from jax.experimental import pallas as pl
from jax.experimental.pallas import tpu as pltpu
from jax.experimental.pallas import tpu_sc as plsc

D = 128
N1 = 10000
NP = 10240
E1 = 320000
G = 256
GP = 512
GM = 272
E2 = 4096

NC = 2
NS = 16
NW = NC * NS

EC = 80
ECH = E1 // EC
NCPW = ECH // NW
EPW = E1 // NW

RPW = NP // NW
RCH = RPW // EC
OB = NP // NS

_MESH = plsc.VectorSubcoreMesh(core_axis_name="c", subcore_axis_name="s")


def _wid():
    return lax.axis_index("s") * NC + lax.axis_index("c")


def _zero_rows(buf, nrows):
    z = jnp.zeros((16,), jnp.float32)

    @pl.loop(0, nrows)
    def _(r):
        for c in range(D // 16):
            buf[r, pl.ds(c * 16, 16)] = z


def _fill_1d(buf, n, val):
    v = jnp.full((16,), val, jnp.float32)

    @pl.loop(0, n // 16)
    def _(k):
        buf[pl.ds(k * 16, 16)] = v


@functools.partial(
    pl.kernel,
    out_type=(
        jax.ShapeDtypeStruct((NP, D), jnp.float32),
        jax.ShapeDtypeStruct((NC, NP), jnp.float32),
        jax.ShapeDtypeStruct((NC, GP), jnp.float32),
        jax.ShapeDtypeStruct((NC, G * G), jnp.float32),
    ),
    mesh=_MESH,
    scratch_types=[
        pltpu.VMEM((RPW,), jnp.int32),
        pltpu.VMEM((EPW,), jnp.int32),
        pltpu.VMEM((RPW,), jnp.int32),
        pltpu.VMEM((64,), jnp.int32),
        pltpu.VMEM((64,), jnp.int32),
        pltpu.VMEM((64,), jnp.int32),
        pltpu.VMEM((128,), jnp.float32),
        pltpu.VMEM((640,), jnp.float32),
        pltpu.VMEM((EC, D), jnp.float32),
        pltpu.VMEM((EC, D), jnp.float32),
        pltpu.VMEM_SHARED((NP,), jnp.float32),
        pltpu.VMEM_SHARED((GP,), jnp.float32),
        pltpu.VMEM_SHARED((G * G,), jnp.float32),
        pltpu.SemaphoreType.DMA,
        pltpu.SemaphoreType.DMA,
        pltpu.SemaphoreType.DMA,
    ],
)
def _sc_prep(emb_hbm, x1_hbm, dst1_hbm, batch_hbm, src2_hbm, dst2_hbm,
             xa_hbm, cnt_hbm, cntg_hbm, a2_hbm,
             gidx, didx, bidx, s2, d2, f2, ones_v, zb,
             rowsA, rowsB, cnt_sh, cntg_sh, a2_sh, semA, semB, ssem):
    cid = lax.axis_index("c")
    sid = lax.axis_index("s")
    wid = sid * NC + cid

    _fill_1d(ones_v, 128, 1.0)
    _fill_1d(zb, 640, 0.0)

    pltpu.sync_copy(zb, cnt_sh.at[pl.ds(sid * OB, OB)])
    pltpu.sync_copy(zb.at[pl.ds(0, GP // NS)],
                    cntg_sh.at[pl.ds(sid * (GP // NS), GP // NS)])
    a2pt = G * G // NS

    @pl.loop(0, 6)
    def _(k):
        pltpu.sync_copy(zb, a2_sh.at[pl.ds(sid * a2pt + k * 640, 640)])

    pltpu.sync_copy(zb.at[pl.ds(0, 256)],
                    a2_sh.at[pl.ds(sid * a2pt + 3840, 256)])

    rbase = wid * RPW
    pltpu.sync_copy(x1_hbm.at[pl.ds(rbase, RPW)], gidx)
    bufs = (rowsA, rowsB)
    sems = (semA, semB)
    pltpu.async_copy(emb_hbm.at[gidx.at[pl.ds(0, EC)]], rowsA, semA)
    for c in range(RCH):
        if c + 1 < RCH:
            pltpu.async_copy(emb_hbm.at[gidx.at[pl.ds((c + 1) * EC, EC)]],
                             bufs[(c + 1) % 2], sems[(c + 1) % 2])
        pltpu.make_async_copy(emb_hbm.at[pl.ds(0, EC)], bufs[c % 2],
                              sems[c % 2]).wait()
        pltpu.sync_copy(bufs[c % 2], xa_hbm.at[pl.ds(rbase + c * EC, EC)])

    plsc.subcore_barrier()

    pltpu.sync_copy(dst1_hbm.at[pl.ds(wid * EPW, EPW)], didx)

    @pl.loop(0, 5)
    def _(blk):
        for j in range(25):
            pltpu.async_copy(
                ones_v.at[pl.ds(0, EC)],
                cnt_sh.at[didx.at[pl.ds((blk * 25 + j) * EC, EC)]],
                ssem, add=True)
        for j in range(25):
            pltpu.make_async_copy(
                ones_v.at[pl.ds(0, EC)],
                cnt_sh.at[didx.at[pl.ds((blk * 25 + j) * EC, EC)]],
                ssem).wait()

    pltpu.sync_copy(batch_hbm.at[pl.ds(wid * RPW, RPW)], bidx)
    for j in range(RCH):
        pltpu.sync_copy(ones_v.at[pl.ds(0, EC)],
                        cntg_sh.at[bidx.at[pl.ds(j * EC, EC)]], add=True)

    e2base = wid * (E2 // NW)
    for j in range(2):
        b = e2base + j * 64
        pltpu.sync_copy(src2_hbm.at[pl.ds(b, 64)], s2)
        pltpu.sync_copy(dst2_hbm.at[pl.ds(b, 64)], d2)
        for c in range(4):
            f2[pl.ds(c * 16, 16)] = (
                d2[pl.ds(c * 16, 16)] * G + s2[pl.ds(c * 16, 16)])
        pltpu.sync_copy(ones_v.at[pl.ds(0, 64)], a2_sh.at[f2], add=True)

    plsc.subcore_barrier()

    pltpu.sync_copy(cnt_sh.at[pl.ds(sid * OB, OB)], zb)
    pltpu.sync_copy(zb, cnt_hbm.at[cid, pl.ds(sid * OB, OB)])

    og = sid * (GP // NS)
    pltpu.sync_copy(cntg_sh.at[pl.ds(og, GP // NS)], zb.at[pl.ds(0, GP // NS)])
    pltpu.sync_copy(zb.at[pl.ds(0, GP // NS)],
                    cntg_hbm.at[cid, pl.ds(og, GP // NS)])

    @pl.loop(0, 6)
    def _(k):
        o = sid * a2pt + k * 640
        pltpu.sync_copy(a2_sh.at[pl.ds(o, 640)], zb)
        pltpu.sync_copy(zb, a2_hbm.at[cid, pl.ds(o, 640)])

    o = sid * a2pt + 3840
    pltpu.sync_copy(a2_sh.at[pl.ds(o, 256)], zb.at[pl.ds(0, 256)])
    pltpu.sync_copy(zb.at[pl.ds(0, 256)], a2_hbm.at[cid, pl.ds(o, 256)])


@functools.partial(
    pl.kernel,
    out_type=jax.ShapeDtypeStruct((NC, NP, D), jnp.float32),
    mesh=_MESH,
    scratch_types=[
        pltpu.VMEM((EPW,), jnp.int32),
        pltpu.VMEM((EPW,), jnp.int32),
        pltpu.VMEM((EC, D), jnp.float32),
        pltpu.VMEM((EC, D), jnp.float32),
        pltpu.VMEM_SHARED((NP, D), jnp.float32),
        pltpu.SemaphoreType.DMA,
        pltpu.SemaphoreType.DMA,
    ],
)
def _sc_agg(x_hbm, src_hbm, dst_hbm, out_hbm,
            sidx, didx, rowsA, rowsB, agg_sh, semA, semB):
    cid = lax.axis_index("c")
    sid = lax.axis_index("s")
    wid = sid * NC + cid

    _zero_rows(rowsA, EC)

    @pl.loop(0, OB // EC)
    def _(k):
        pltpu.sync_copy(rowsA, agg_sh.at[pl.ds(sid * OB + k * EC, EC)])

    pltpu.sync_copy(src_hbm.at[pl.ds(wid * EPW, EPW)], sidx)
    pltpu.sync_copy(dst_hbm.at[pl.ds(wid * EPW, EPW)], didx)
    plsc.subcore_barrier()

    def gath(c, buf, sem):
        pltpu.async_copy(x_hbm.at[sidx.at[pl.ds(c * EC, EC)]], buf, sem)

    def gwait(buf, sem):
        pltpu.make_async_copy(x_hbm.at[pl.ds(0, EC)], buf, sem).wait()

    def scat(c, buf):
        pltpu.sync_copy(buf, agg_sh.at[didx.at[pl.ds(c * EC, EC)]], add=True)

    gath(0, rowsA, semA)
    for k in range(OB // EC):
        pltpu.async_copy(rowsB, agg_sh.at[pl.ds(sid * OB + k * EC, EC)], semB)
    for k in range(OB // EC):
        pltpu.make_async_copy(rowsB, agg_sh.at[pl.ds(sid * OB + k * EC, EC)],
                              semB).wait()
    plsc.subcore_barrier()

    @pl.loop(0, (NCPW - 1) // 2)
    def _(j2):
        c = 2 * j2
        gath(c + 1, rowsB, semB)
        gwait(rowsA, semA)
        scat(c, rowsA)
        gath(c + 2, rowsA, semA)
        gwait(rowsB, semB)
        scat(c + 1, rowsB)

    gwait(rowsA, semA)
    scat(NCPW - 1, rowsA)

    plsc.subcore_barrier()

    obufs = (rowsA, rowsB)
    osems = (semA, semB)
    for k in range(OB // EC):
        o = sid * OB + k * EC
        b, sm = obufs[k % 2], osems[k % 2]
        if k >= 2:
            op = sid * OB + (k - 2) * EC
            pltpu.make_async_copy(b, out_hbm.at[cid, pl.ds(op, EC)], sm).wait()
        pltpu.sync_copy(agg_sh.at[pl.ds(o, EC)], b)
        pltpu.async_copy(b, out_hbm.at[cid, pl.ds(o, EC)], sm)
    for k in (OB // EC - 2, OB // EC - 1):
        o = sid * OB + k * EC
        pltpu.make_async_copy(obufs[k % 2], out_hbm.at[cid, pl.ds(o, EC)],
                              osems[k % 2]).wait()


@functools.partial(
    pl.kernel,
    out_type=(
        jax.ShapeDtypeStruct((NC, GP, D), jnp.float32),
        jax.ShapeDtypeStruct((NW, GM, D), jnp.float32),
    ),
    mesh=_MESH,
    scratch_types=[
        pltpu.VMEM((RPW,), jnp.int32),
        pltpu.VMEM((RPW, D), jnp.float32),
        pltpu.VMEM((GM, D), jnp.float32),
        pltpu.VMEM_SHARED((GP, D), jnp.float32),
        pltpu.SemaphoreType.DMA,
    ],
)
def _sc_pool(x_hbm, batch_hbm, sum_hbm, max_hbm, bidx, rows, lmax, sum_sh, sem):
    cid = lax.axis_index("c")
    sid = lax.axis_index("s")
    wid = sid * NC + cid
    base = wid * RPW

    pltpu.async_copy(batch_hbm.at[pl.ds(base, RPW)], bidx, sem)
    pltpu.async_copy(x_hbm.at[pl.ds(base, RPW)], rows, sem)
    _zero_rows(lmax, GM)
    pltpu.sync_copy(lmax.at[pl.ds(0, GP // NS)],
                    sum_sh.at[pl.ds(sid * (GP // NS), GP // NS)])
    pltpu.make_async_copy(batch_hbm.at[pl.ds(base, RPW)], bidx, sem).wait()
    pltpu.make_async_copy(x_hbm.at[pl.ds(base, RPW)], rows, sem).wait()
    plsc.subcore_barrier()

    for j in range(RCH):
        pltpu.async_copy(rows.at[pl.ds(j * EC, EC)],
                         sum_sh.at[bidx.at[pl.ds(j * EC, EC)]], sem, add=True)

    @pl.loop(0, RPW // 16)
    def _(q):
        bvec = bidx[pl.ds(q * 16, 16)]
        for r in range(16):
            g = bvec[r]
            for c in range(D // 16):
                sl = pl.ds(c * 16, 16)
                lmax[g, sl] = jnp.maximum(lmax[g, sl],
                                          rows[q * 16 + r, sl])

    pltpu.sync_copy(lmax, max_hbm.at[wid])

    for j in range(RCH):
        pltpu.make_async_copy(rows.at[pl.ds(j * EC, EC)],
                              sum_sh.at[bidx.at[pl.ds(j * EC, EC)]], sem).wait()
    plsc.subcore_barrier()
    o = sid * (GP // NS)
    pltpu.sync_copy(sum_sh.at[pl.ds(o, GP // NS)], lmax.at[pl.ds(0, GP // NS)])
    pltpu.sync_copy(lmax.at[pl.ds(0, GP // NS)],
                    sum_hbm.at[cid, pl.ds(o, GP // NS)])


_RB = 1024


def _tc_layer_body(aggp_ref, x_ref, cntp_ref, wl_ref, wr_ref, b_ref, o_ref):
    i = pl.program_id(0)
    cnt = cntp_ref[0, pl.ds(i * _RB, _RB)] + cntp_ref[1, pl.ds(i * _RB, _RB)]
    inv = 1.0 / jnp.maximum(cnt, 1.0)
    agg = (aggp_ref[0] + aggp_ref[1]) * inv.reshape(_RB, 1)
    acc = lax.dot_general(agg, wl_ref[...], (((1,), (1,)), ((), ())),
                          preferred_element_type=jnp.float32)
    acc += lax.dot_general(x_ref[...], wr_ref[...], (((1,), (1,)), ((), ())),
                           preferred_element_type=jnp.float32)
    o_ref[...] = jnp.maximum(acc + b_ref[...], 0.0)


def _tc_layer(aggp, x, cntp, wl, wr, b2d):
    return pl.pallas_call(
        _tc_layer_body,
        grid=(NP // _RB,),
        in_specs=[
            pl.BlockSpec((NC, _RB, D), lambda i: (0, i, 0)),
            pl.BlockSpec((_RB, D), lambda i: (i, 0)),
            pl.BlockSpec((NC, NP), lambda i: (0, 0)),
            pl.BlockSpec((D, D), lambda i: (0, 0)),
            pl.BlockSpec((D, D), lambda i: (0, 0)),
            pl.BlockSpec((1, D), lambda i: (0, 0)),
        ],
        out_specs=pl.BlockSpec((_RB, D), lambda i: (i, 0)),
        out_shape=jax.ShapeDtypeStruct((NP, D), jnp.float32),
    )(aggp, x, cntp, wl, wr, b2d)


def _tc_final_body(sump_ref, maxp_ref, cntgp_ref, a2p_ref,
                   lin1w_ref, lin1b_ref,
                   wl0_ref, wr0_ref, wd0_ref, b0_ref,
                   wl1_ref, wr1_ref, wd1_ref, b1_ref, o_ref):
    gm = maxp_ref[0, 0:G, :]
    for k in range(1, NW):
        gm = jnp.maximum(gm, maxp_ref[k, 0:G, :])
    ga = sump_ref[0, 0:G, :] + sump_ref[1, 0:G, :]
    cg = cntgp_ref[0, pl.ds(0, G)] + cntgp_ref[1, pl.ds(0, G)]
    ga = ga * (1.0 / jnp.maximum(cg, 1.0)).reshape(G, 1)
    xcat = jnp.concatenate([gm, ga], axis=1)
    xdrug = lax.dot_general(xcat, lin1w_ref[...], (((1,), (1,)), ((), ())),
                            preferred_element_type=jnp.float32)
    xdrug = jnp.maximum(xdrug + lin1b_ref[...], 0.0)

    a2 = a2p_ref[0] + a2p_ref[1]
    cnt2 = jnp.sum(a2, axis=1, keepdims=True)
    an = a2 / jnp.maximum(cnt2, 1.0)

    x2 = xdrug
    for (wl, wr, wd, b) in ((wl0_ref, wr0_ref, wd0_ref, b0_ref),
                            (wl1_ref, wr1_ref, wd1_ref, b1_ref)):
        agg2 = lax.dot_general(an, x2, (((1,), (0,)), ((), ())),
                               preferred_element_type=jnp.float32)
        acc = lax.dot_general(agg2, wl[...], (((1,), (1,)), ((), ())),
                              preferred_element_type=jnp.float32)
        acc += lax.dot_general(x2, wr[...], (((1,), (1,)), ((), ())),
                               preferred_element_type=jnp.float32)
        acc += lax.dot_general(xdrug, wd[...], (((1,), (1,)), ((), ())),
                               preferred_element_type=jnp.float32)
        x2 = jnp.maximum(acc + b[...], 0.0)
    o_ref[...] = x2


def _tc_final(sump, maxp, cntgp, a2p, lin1w, lin1b2d,
              wl0, wr0, wd0, b02d, wl1, wr1, wd1, b12d):
    return pl.pallas_call(
        _tc_final_body,
        out_shape=jax.ShapeDtypeStruct((G, D), jnp.float32),
    )(sump, maxp, cntgp, a2p, lin1w, lin1b2d,
      wl0, wr0, wd0, b02d, wl1, wr1, wd1, b12d)


def kernel(emb1, emb2, Wl1_0, Wr1_0, b1_0, Wl1_1, Wr1_1, b1_1, lin1_W, lin1_b,
           Wl2_0, Wr2_0, Wd2_0, b2_0, Wl2_1, Wr2_1, Wd2_1, b2_1,
           x1, edge_index1, batch1, x2_idx, edge_index2):
    x1p = jnp.concatenate([x1, jnp.full((NP - N1,), N1, jnp.int32)])
    batchp = jnp.concatenate([batch1, jnp.full((NP - N1,), G, jnp.int32)])
    src1 = edge_index1[0]
    dst1 = edge_index1[1]
    src2, dst2 = edge_index2[0], edge_index2[1]

    xA, cntp, cntgp, a2p_flat = _sc_prep(emb1, x1p, dst1, batchp, src2, dst2)
    a2p = a2p_flat.reshape(NC, G, G)

    aggp = _sc_agg(xA, src1, dst1)
    xA = _tc_layer(aggp, xA, cntp, Wl1_0, Wr1_0, b1_0.reshape(1, D))
    aggp = _sc_agg(xA, src1, dst1)
    xA = _tc_layer(aggp, xA, cntp, Wl1_1, Wr1_1, b1_1.reshape(1, D))

    sump, maxp = _sc_pool(xA, batchp)

    return _tc_final(sump, maxp, cntgp, a2p, lin1_W, lin1_b.reshape(1, D),
                     Wl2_0, Wr2_0, Wd2_0, b2_0.reshape(1, D),
                     Wl2_1, Wr2_1, Wd2_1, b2_1.reshape(1, D))


# speedup vs baseline: 11.0036x; 1.0098x over previous
import functools

import jax
import jax.numpy as jnp
from jax import lax
from jax.experimental import pallas as pl
from jax.experimental.pallas import tpu as pltpu
from jax.experimental.pallas import tpu_sc as plsc

D = 128
N1 = 10000
NP = 10240
E1 = 320000
G = 256
GP = 512
GM = 272
E2 = 4096

NC = 2
NS = 16
NW = NC * NS

EC = 80
ECH = E1 // EC
NCPW = ECH // NW
EPW = E1 // NW

RPW = NP // NW
RCH = RPW // EC
OB = NP // NS

_MESH = plsc.VectorSubcoreMesh(core_axis_name="c", subcore_axis_name="s")


def _wid():
    return lax.axis_index("s") * NC + lax.axis_index("c")


def _zero_rows(buf, nrows):
    z = jnp.zeros((16,), jnp.float32)

    @pl.loop(0, nrows)
    def _(r):
        for c in range(D // 16):
            buf[r, pl.ds(c * 16, 16)] = z


def _fill_1d(buf, n, val):
    v = jnp.full((16,), val, jnp.float32)

    @pl.loop(0, n // 16)
    def _(k):
        buf[pl.ds(k * 16, 16)] = v


@functools.partial(
    pl.kernel,
    out_type=(
        jax.ShapeDtypeStruct((NP, D), jnp.float32),
        jax.ShapeDtypeStruct((NC, NP), jnp.float32),
        jax.ShapeDtypeStruct((NC, GP), jnp.float32),
        jax.ShapeDtypeStruct((NC, G * G), jnp.float32),
    ),
    mesh=_MESH,
    scratch_types=[
        pltpu.VMEM((RPW,), jnp.int32),
        pltpu.VMEM((EPW,), jnp.int32),
        pltpu.VMEM((RPW,), jnp.int32),
        pltpu.VMEM((64,), jnp.int32),
        pltpu.VMEM((64,), jnp.int32),
        pltpu.VMEM((64,), jnp.int32),
        pltpu.VMEM((128,), jnp.float32),
        pltpu.VMEM((640,), jnp.float32),
        pltpu.VMEM((EC, D), jnp.float32),
        pltpu.VMEM((EC, D), jnp.float32),
        pltpu.VMEM_SHARED((NP,), jnp.float32),
        pltpu.VMEM_SHARED((GP,), jnp.float32),
        pltpu.VMEM_SHARED((G * G,), jnp.float32),
        pltpu.SemaphoreType.DMA,
        pltpu.SemaphoreType.DMA,
        pltpu.SemaphoreType.DMA,
    ],
)
def _sc_prep(emb_hbm, x1_hbm, dst1_hbm, batch_hbm, src2_hbm, dst2_hbm,
             xa_hbm, cnt_hbm, cntg_hbm, a2_hbm,
             gidx, didx, bidx, s2, d2, f2, ones_v, zb,
             rowsA, rowsB, cnt_sh, cntg_sh, a2_sh, semA, semB, ssem):
    cid = lax.axis_index("c")
    sid = lax.axis_index("s")
    wid = sid * NC + cid

    _fill_1d(ones_v, 128, 1.0)
    _fill_1d(zb, 640, 0.0)

    pltpu.sync_copy(zb, cnt_sh.at[pl.ds(sid * OB, OB)])
    pltpu.sync_copy(zb.at[pl.ds(0, GP // NS)],
                    cntg_sh.at[pl.ds(sid * (GP // NS), GP // NS)])
    a2pt = G * G // NS

    @pl.loop(0, 6)
    def _(k):
        pltpu.sync_copy(zb, a2_sh.at[pl.ds(sid * a2pt + k * 640, 640)])

    pltpu.sync_copy(zb.at[pl.ds(0, 256)],
                    a2_sh.at[pl.ds(sid * a2pt + 3840, 256)])

    rbase = wid * RPW
    pltpu.sync_copy(x1_hbm.at[pl.ds(rbase, RPW)], gidx)
    bufs = (rowsA, rowsB)
    sems = (semA, semB)
    pltpu.async_copy(emb_hbm.at[gidx.at[pl.ds(0, EC)]], rowsA, semA)
    for c in range(RCH):
        if c + 1 < RCH:
            pltpu.async_copy(emb_hbm.at[gidx.at[pl.ds((c + 1) * EC, EC)]],
                             bufs[(c + 1) % 2], sems[(c + 1) % 2])
        pltpu.make_async_copy(emb_hbm.at[pl.ds(0, EC)], bufs[c % 2],
                              sems[c % 2]).wait()
        pltpu.sync_copy(bufs[c % 2], xa_hbm.at[pl.ds(rbase + c * EC, EC)])

    plsc.subcore_barrier()

    pltpu.sync_copy(dst1_hbm.at[pl.ds(wid * EPW, EPW)], didx)

    @pl.loop(0, 5)
    def _(blk):
        for j in range(25):
            pltpu.async_copy(
                ones_v.at[pl.ds(0, EC)],
                cnt_sh.at[didx.at[pl.ds((blk * 25 + j) * EC, EC)]],
                ssem, add=True)
        for j in range(25):
            pltpu.make_async_copy(
                ones_v.at[pl.ds(0, EC)],
                cnt_sh.at[didx.at[pl.ds((blk * 25 + j) * EC, EC)]],
                ssem).wait()

    pltpu.sync_copy(batch_hbm.at[pl.ds(wid * RPW, RPW)], bidx)
    for j in range(RCH):
        pltpu.sync_copy(ones_v.at[pl.ds(0, EC)],
                        cntg_sh.at[bidx.at[pl.ds(j * EC, EC)]], add=True)

    e2base = wid * (E2 // NW)
    for j in range(2):
        b = e2base + j * 64
        pltpu.sync_copy(src2_hbm.at[pl.ds(b, 64)], s2)
        pltpu.sync_copy(dst2_hbm.at[pl.ds(b, 64)], d2)
        for c in range(4):
            f2[pl.ds(c * 16, 16)] = (
                d2[pl.ds(c * 16, 16)] * G + s2[pl.ds(c * 16, 16)])
        pltpu.sync_copy(ones_v.at[pl.ds(0, 64)], a2_sh.at[f2], add=True)

    plsc.subcore_barrier()

    pltpu.sync_copy(cnt_sh.at[pl.ds(sid * OB, OB)], zb)
    pltpu.sync_copy(zb, cnt_hbm.at[cid, pl.ds(sid * OB, OB)])

    og = sid * (GP // NS)
    pltpu.sync_copy(cntg_sh.at[pl.ds(og, GP // NS)], zb.at[pl.ds(0, GP // NS)])
    pltpu.sync_copy(zb.at[pl.ds(0, GP // NS)],
                    cntg_hbm.at[cid, pl.ds(og, GP // NS)])

    @pl.loop(0, 6)
    def _(k):
        o = sid * a2pt + k * 640
        pltpu.sync_copy(a2_sh.at[pl.ds(o, 640)], zb)
        pltpu.sync_copy(zb, a2_hbm.at[cid, pl.ds(o, 640)])

    o = sid * a2pt + 3840
    pltpu.sync_copy(a2_sh.at[pl.ds(o, 256)], zb.at[pl.ds(0, 256)])
    pltpu.sync_copy(zb.at[pl.ds(0, 256)], a2_hbm.at[cid, pl.ds(o, 256)])


@functools.partial(
    pl.kernel,
    out_type=jax.ShapeDtypeStruct((NC, NP, D), jnp.float32),
    mesh=_MESH,
    scratch_types=[
        pltpu.VMEM((EPW,), jnp.int32),
        pltpu.VMEM((EPW,), jnp.int32),
        pltpu.VMEM((EC, D), jnp.float32),
        pltpu.VMEM((EC, D), jnp.float32),
        pltpu.VMEM_SHARED((NP, D), jnp.float32),
        pltpu.SemaphoreType.DMA,
        pltpu.SemaphoreType.DMA,
    ],
)
def _sc_agg(x_hbm, src_hbm, dst_hbm, out_hbm,
            sidx, didx, rowsA, rowsB, agg_sh, semA, semB):
    cid = lax.axis_index("c")
    sid = lax.axis_index("s")
    wid = sid * NC + cid

    _zero_rows(rowsA, EC)

    @pl.loop(0, OB // EC)
    def _(k):
        pltpu.sync_copy(rowsA, agg_sh.at[pl.ds(sid * OB + k * EC, EC)])

    pltpu.sync_copy(src_hbm.at[pl.ds(wid * EPW, EPW)], sidx)
    pltpu.sync_copy(dst_hbm.at[pl.ds(wid * EPW, EPW)], didx)
    plsc.subcore_barrier()

    def gath(c, buf, sem):
        pltpu.async_copy(x_hbm.at[sidx.at[pl.ds(c * EC, EC)]], buf, sem)

    def gwait(buf, sem):
        pltpu.make_async_copy(x_hbm.at[pl.ds(0, EC)], buf, sem).wait()

    def scat(c, buf):
        pltpu.sync_copy(buf, agg_sh.at[didx.at[pl.ds(c * EC, EC)]], add=True)

    gath(0, rowsA, semA)

    @pl.loop(0, (NCPW - 1) // 2)
    def _(j2):
        c = 2 * j2
        gath(c + 1, rowsB, semB)
        gwait(rowsA, semA)
        scat(c, rowsA)
        gath(c + 2, rowsA, semA)
        gwait(rowsB, semB)
        scat(c + 1, rowsB)

    gwait(rowsA, semA)
    scat(NCPW - 1, rowsA)

    plsc.subcore_barrier()

    obufs = (rowsA, rowsB)
    osems = (semA, semB)
    for k in range(OB // EC):
        o = sid * OB + k * EC
        b, sm = obufs[k % 2], osems[k % 2]
        if k >= 2:
            op = sid * OB + (k - 2) * EC
            pltpu.make_async_copy(b, out_hbm.at[cid, pl.ds(op, EC)], sm).wait()
        pltpu.sync_copy(agg_sh.at[pl.ds(o, EC)], b)
        pltpu.async_copy(b, out_hbm.at[cid, pl.ds(o, EC)], sm)
    for k in (OB // EC - 2, OB // EC - 1):
        o = sid * OB + k * EC
        pltpu.make_async_copy(obufs[k % 2], out_hbm.at[cid, pl.ds(o, EC)],
                              osems[k % 2]).wait()


@functools.partial(
    pl.kernel,
    out_type=(
        jax.ShapeDtypeStruct((NC, GP, D), jnp.float32),
        jax.ShapeDtypeStruct((NW, GM, D), jnp.float32),
    ),
    mesh=_MESH,
    scratch_types=[
        pltpu.VMEM((RPW,), jnp.int32),
        pltpu.VMEM((RPW, D), jnp.float32),
        pltpu.VMEM((GM, D), jnp.float32),
        pltpu.VMEM_SHARED((GP, D), jnp.float32),
        pltpu.SemaphoreType.DMA,
    ],
)
def _sc_pool(x_hbm, batch_hbm, sum_hbm, max_hbm, bidx, rows, lmax, sum_sh, sem):
    cid = lax.axis_index("c")
    sid = lax.axis_index("s")
    wid = sid * NC + cid
    base = wid * RPW

    pltpu.async_copy(batch_hbm.at[pl.ds(base, RPW)], bidx, sem)
    pltpu.async_copy(x_hbm.at[pl.ds(base, RPW)], rows, sem)
    _zero_rows(lmax, GM)
    pltpu.sync_copy(lmax.at[pl.ds(0, GP // NS)],
                    sum_sh.at[pl.ds(sid * (GP // NS), GP // NS)])
    pltpu.make_async_copy(batch_hbm.at[pl.ds(base, RPW)], bidx, sem).wait()
    pltpu.make_async_copy(x_hbm.at[pl.ds(base, RPW)], rows, sem).wait()
    plsc.subcore_barrier()

    for j in range(RCH):
        pltpu.async_copy(rows.at[pl.ds(j * EC, EC)],
                         sum_sh.at[bidx.at[pl.ds(j * EC, EC)]], sem, add=True)

    @pl.loop(0, RPW // 16)
    def _(q):
        bvec = bidx[pl.ds(q * 16, 16)]
        for r in range(16):
            g = bvec[r]
            for c in range(D // 16):
                sl = pl.ds(c * 16, 16)
                lmax[g, sl] = jnp.maximum(lmax[g, sl],
                                          rows[q * 16 + r, sl])

    pltpu.sync_copy(lmax, max_hbm.at[wid])

    for j in range(RCH):
        pltpu.make_async_copy(rows.at[pl.ds(j * EC, EC)],
                              sum_sh.at[bidx.at[pl.ds(j * EC, EC)]], sem).wait()
    plsc.subcore_barrier()
    o = sid * (GP // NS)
    pltpu.sync_copy(sum_sh.at[pl.ds(o, GP // NS)], lmax.at[pl.ds(0, GP // NS)])
    pltpu.sync_copy(lmax.at[pl.ds(0, GP // NS)],
                    sum_hbm.at[cid, pl.ds(o, GP // NS)])


_RB = 1024


def _tc_layer_body(aggp_ref, x_ref, cntp_ref, wl_ref, wr_ref, b_ref, o_ref):
    i = pl.program_id(0)
    cnt = cntp_ref[0, pl.ds(i * _RB, _RB)] + cntp_ref[1, pl.ds(i * _RB, _RB)]
    inv = 1.0 / jnp.maximum(cnt, 1.0)
    agg = (aggp_ref[0] + aggp_ref[1]) * inv.reshape(_RB, 1)
    acc = lax.dot_general(agg, wl_ref[...], (((1,), (1,)), ((), ())),
                          preferred_element_type=jnp.float32)
    acc += lax.dot_general(x_ref[...], wr_ref[...], (((1,), (1,)), ((), ())),
                           preferred_element_type=jnp.float32)
    o_ref[...] = jnp.maximum(acc + b_ref[...], 0.0)


def _tc_layer(aggp, x, cntp, wl, wr, b2d):
    return pl.pallas_call(
        _tc_layer_body,
        grid=(NP // _RB,),
        in_specs=[
            pl.BlockSpec((NC, _RB, D), lambda i: (0, i, 0)),
            pl.BlockSpec((_RB, D), lambda i: (i, 0)),
            pl.BlockSpec((NC, NP), lambda i: (0, 0)),
            pl.BlockSpec((D, D), lambda i: (0, 0)),
            pl.BlockSpec((D, D), lambda i: (0, 0)),
            pl.BlockSpec((1, D), lambda i: (0, 0)),
        ],
        out_specs=pl.BlockSpec((_RB, D), lambda i: (i, 0)),
        out_shape=jax.ShapeDtypeStruct((NP, D), jnp.float32),
    )(aggp, x, cntp, wl, wr, b2d)


def _tc_final_body(sump_ref, maxp_ref, cntgp_ref, a2p_ref,
                   lin1w_ref, lin1b_ref,
                   wl0_ref, wr0_ref, wd0_ref, b0_ref,
                   wl1_ref, wr1_ref, wd1_ref, b1_ref, o_ref):
    gm = maxp_ref[0, 0:G, :]
    for k in range(1, NW):
        gm = jnp.maximum(gm, maxp_ref[k, 0:G, :])
    ga = sump_ref[0, 0:G, :] + sump_ref[1, 0:G, :]
    cg = cntgp_ref[0, pl.ds(0, G)] + cntgp_ref[1, pl.ds(0, G)]
    ga = ga * (1.0 / jnp.maximum(cg, 1.0)).reshape(G, 1)
    xcat = jnp.concatenate([gm, ga], axis=1)
    xdrug = lax.dot_general(xcat, lin1w_ref[...], (((1,), (1,)), ((), ())),
                            preferred_element_type=jnp.float32)
    xdrug = jnp.maximum(xdrug + lin1b_ref[...], 0.0)

    a2 = a2p_ref[0] + a2p_ref[1]
    cnt2 = jnp.sum(a2, axis=1, keepdims=True)
    an = a2 / jnp.maximum(cnt2, 1.0)

    x2 = xdrug
    for (wl, wr, wd, b) in ((wl0_ref, wr0_ref, wd0_ref, b0_ref),
                            (wl1_ref, wr1_ref, wd1_ref, b1_ref)):
        agg2 = lax.dot_general(an, x2, (((1,), (0,)), ((), ())),
                               preferred_element_type=jnp.float32)
        acc = lax.dot_general(agg2, wl[...], (((1,), (1,)), ((), ())),
                              preferred_element_type=jnp.float32)
        acc += lax.dot_general(x2, wr[...], (((1,), (1,)), ((), ())),
                               preferred_element_type=jnp.float32)
        acc += lax.dot_general(xdrug, wd[...], (((1,), (1,)), ((), ())),
                               preferred_element_type=jnp.float32)
        x2 = jnp.maximum(acc + b[...], 0.0)
    o_ref[...] = x2


def _tc_final(sump, maxp, cntgp, a2p, lin1w, lin1b2d,
              wl0, wr0, wd0, b02d, wl1, wr1, wd1, b12d):
    return pl.pallas_call(
        _tc_final_body,
        out_shape=jax.ShapeDtypeStruct((G, D), jnp.float32),
    )(sump, maxp, cntgp, a2p, lin1w, lin1b2d,
      wl0, wr0, wd0, b02d, wl1, wr1, wd1, b12d)


def kernel(emb1, emb2, Wl1_0, Wr1_0, b1_0, Wl1_1, Wr1_1, b1_1, lin1_W, lin1_b,
           Wl2_0, Wr2_0, Wd2_0, b2_0, Wl2_1, Wr2_1, Wd2_1, b2_1,
           x1, edge_index1, batch1, x2_idx, edge_index2):
    x1p = jnp.concatenate([x1, jnp.full((NP - N1,), N1, jnp.int32)])
    batchp = jnp.concatenate([batch1, jnp.full((NP - N1,), G, jnp.int32)])
    src1 = edge_index1[0]
    dst1 = edge_index1[1]
    src2, dst2 = edge_index2[0], edge_index2[1]

    xA, cntp, cntgp, a2p_flat = _sc_prep(emb1, x1p, dst1, batchp, src2, dst2)
    a2p = a2p_flat.reshape(NC, G, G)

    aggp = _sc_agg(xA, src1, dst1)
    xA = _tc_layer(aggp, xA, cntp, Wl1_0, Wr1_0, b1_0.reshape(1, D))
    aggp = _sc_agg(xA, src1, dst1)
    xA = _tc_layer(aggp, xA, cntp, Wl1_1, Wr1_1, b1_1.reshape(1, D))

    sump, maxp = _sc_pool(xA, batchp)

    return _tc_final(sump, maxp, cntgp, a2p, lin1_W, lin1_b.reshape(1, D),
                     Wl2_0, Wr2_0, Wd2_0, b2_0.reshape(1, D),
                     Wl2_1, Wr2_1, Wd2_1, b2_1.reshape(1, D))


# speedup vs baseline: 11.1596x; 1.0142x over previous
import functools

import jax
import jax.numpy as jnp
from jax import lax
from jax.experimental import pallas as pl
from jax.experimental.pallas import tpu as pltpu
from jax.experimental.pallas import tpu_sc as plsc

D = 128
N1 = 10000
NP = 10240
E1 = 320000
G = 256
GP = 512
GM = 272
E2 = 4096

NC = 2
NS = 16
NW = NC * NS

EC = 80
ECH = E1 // EC
NCPW = ECH // NW
EPW = E1 // NW

RPW = NP // NW
RCH = RPW // EC
OB = NP // NS

_MESH = plsc.VectorSubcoreMesh(core_axis_name="c", subcore_axis_name="s")


def _wid():
    return lax.axis_index("s") * NC + lax.axis_index("c")


def _zero_rows(buf, nrows):
    z = jnp.zeros((16,), jnp.float32)

    @pl.loop(0, nrows)
    def _(r):
        for c in range(D // 16):
            buf[r, pl.ds(c * 16, 16)] = z


def _fill_1d(buf, n, val):
    v = jnp.full((16,), val, jnp.float32)

    @pl.loop(0, n // 16)
    def _(k):
        buf[pl.ds(k * 16, 16)] = v


@functools.partial(
    pl.kernel,
    out_type=(
        jax.ShapeDtypeStruct((NP, D), jnp.float32),
        jax.ShapeDtypeStruct((NC, NP), jnp.float32),
        jax.ShapeDtypeStruct((NC, GP), jnp.float32),
        jax.ShapeDtypeStruct((NC, G * G), jnp.float32),
    ),
    mesh=_MESH,
    scratch_types=[
        pltpu.VMEM((RPW,), jnp.int32),
        pltpu.VMEM((EPW,), jnp.int32),
        pltpu.VMEM((RPW,), jnp.int32),
        pltpu.VMEM((64,), jnp.int32),
        pltpu.VMEM((64,), jnp.int32),
        pltpu.VMEM((64,), jnp.int32),
        pltpu.VMEM((128,), jnp.float32),
        pltpu.VMEM((640,), jnp.float32),
        pltpu.VMEM((EC, D), jnp.float32),
        pltpu.VMEM((EC, D), jnp.float32),
        pltpu.VMEM_SHARED((NP,), jnp.float32),
        pltpu.VMEM_SHARED((GP,), jnp.float32),
        pltpu.VMEM_SHARED((G * G,), jnp.float32),
        pltpu.SemaphoreType.DMA,
        pltpu.SemaphoreType.DMA,
        pltpu.SemaphoreType.DMA,
    ],
)
def _sc_prep(emb_hbm, x1_hbm, dst1_hbm, batch_hbm, src2_hbm, dst2_hbm,
             xa_hbm, cnt_hbm, cntg_hbm, a2_hbm,
             gidx, didx, bidx, s2, d2, f2, ones_v, zb,
             rowsA, rowsB, cnt_sh, cntg_sh, a2_sh, semA, semB, ssem):
    cid = lax.axis_index("c")
    sid = lax.axis_index("s")
    wid = sid * NC + cid

    _fill_1d(ones_v, 128, 1.0)
    _fill_1d(zb, 640, 0.0)

    pltpu.sync_copy(zb, cnt_sh.at[pl.ds(sid * OB, OB)])
    pltpu.sync_copy(zb.at[pl.ds(0, GP // NS)],
                    cntg_sh.at[pl.ds(sid * (GP // NS), GP // NS)])
    a2pt = G * G // NS

    @pl.loop(0, 6)
    def _(k):
        pltpu.sync_copy(zb, a2_sh.at[pl.ds(sid * a2pt + k * 640, 640)])

    pltpu.sync_copy(zb.at[pl.ds(0, 256)],
                    a2_sh.at[pl.ds(sid * a2pt + 3840, 256)])

    rbase = wid * RPW
    pltpu.sync_copy(x1_hbm.at[pl.ds(rbase, RPW)], gidx)
    bufs = (rowsA, rowsB)
    sems = (semA, semB)
    pltpu.async_copy(emb_hbm.at[gidx.at[pl.ds(0, EC)]], rowsA, semA)
    for c in range(RCH):
        if c + 1 < RCH:
            pltpu.async_copy(emb_hbm.at[gidx.at[pl.ds((c + 1) * EC, EC)]],
                             bufs[(c + 1) % 2], sems[(c + 1) % 2])
        pltpu.make_async_copy(emb_hbm.at[pl.ds(0, EC)], bufs[c % 2],
                              sems[c % 2]).wait()
        pltpu.sync_copy(bufs[c % 2], xa_hbm.at[pl.ds(rbase + c * EC, EC)])

    plsc.subcore_barrier()

    pltpu.sync_copy(dst1_hbm.at[pl.ds(wid * EPW, EPW)], didx)

    @pl.loop(0, 5)
    def _(blk):
        for j in range(25):
            pltpu.async_copy(
                ones_v.at[pl.ds(0, EC)],
                cnt_sh.at[didx.at[pl.ds((blk * 25 + j) * EC, EC)]],
                ssem, add=True)
        for j in range(25):
            pltpu.make_async_copy(
                ones_v.at[pl.ds(0, EC)],
                cnt_sh.at[didx.at[pl.ds((blk * 25 + j) * EC, EC)]],
                ssem).wait()

    pltpu.sync_copy(batch_hbm.at[pl.ds(wid * RPW, RPW)], bidx)
    for j in range(RCH):
        pltpu.sync_copy(ones_v.at[pl.ds(0, EC)],
                        cntg_sh.at[bidx.at[pl.ds(j * EC, EC)]], add=True)

    e2base = wid * (E2 // NW)
    for j in range(2):
        b = e2base + j * 64
        pltpu.sync_copy(src2_hbm.at[pl.ds(b, 64)], s2)
        pltpu.sync_copy(dst2_hbm.at[pl.ds(b, 64)], d2)
        for c in range(4):
            f2[pl.ds(c * 16, 16)] = (
                d2[pl.ds(c * 16, 16)] * G + s2[pl.ds(c * 16, 16)])
        pltpu.sync_copy(ones_v.at[pl.ds(0, 64)], a2_sh.at[f2], add=True)

    plsc.subcore_barrier()

    pltpu.sync_copy(cnt_sh.at[pl.ds(sid * OB, OB)], zb)
    pltpu.sync_copy(zb, cnt_hbm.at[cid, pl.ds(sid * OB, OB)])

    og = sid * (GP // NS)
    pltpu.sync_copy(cntg_sh.at[pl.ds(og, GP // NS)], zb.at[pl.ds(0, GP // NS)])
    pltpu.sync_copy(zb.at[pl.ds(0, GP // NS)],
                    cntg_hbm.at[cid, pl.ds(og, GP // NS)])

    @pl.loop(0, 6)
    def _(k):
        o = sid * a2pt + k * 640
        pltpu.sync_copy(a2_sh.at[pl.ds(o, 640)], zb)
        pltpu.sync_copy(zb, a2_hbm.at[cid, pl.ds(o, 640)])

    o = sid * a2pt + 3840
    pltpu.sync_copy(a2_sh.at[pl.ds(o, 256)], zb.at[pl.ds(0, 256)])
    pltpu.sync_copy(zb.at[pl.ds(0, 256)], a2_hbm.at[cid, pl.ds(o, 256)])


@functools.partial(
    pl.kernel,
    out_type=jax.ShapeDtypeStruct((NC, NP, D), jnp.float32),
    mesh=_MESH,
    scratch_types=[
        pltpu.VMEM((EPW,), jnp.int32),
        pltpu.VMEM((EPW,), jnp.int32),
        pltpu.VMEM((EC, D), jnp.float32),
        pltpu.VMEM((EC, D), jnp.float32),
        pltpu.VMEM_SHARED((NP, D), jnp.float32),
        pltpu.SemaphoreType.DMA,
        pltpu.SemaphoreType.DMA,
    ],
)
def _sc_agg(x_hbm, src_hbm, dst_hbm, out_hbm,
            sidx, didx, rowsA, rowsB, agg_sh, semA, semB):
    cid = lax.axis_index("c")
    sid = lax.axis_index("s")
    wid = sid * NC + cid

    pltpu.async_copy(src_hbm.at[pl.ds(wid * EPW, EPW)], sidx, semA)
    pltpu.async_copy(dst_hbm.at[pl.ds(wid * EPW, EPW)], didx, semA)

    _zero_rows(rowsA, EC)

    @pl.loop(0, OB // EC)
    def _(k):
        pltpu.sync_copy(rowsA, agg_sh.at[pl.ds(sid * OB + k * EC, EC)])

    pltpu.make_async_copy(src_hbm.at[pl.ds(wid * EPW, EPW)], sidx, semA).wait()
    pltpu.make_async_copy(dst_hbm.at[pl.ds(wid * EPW, EPW)], didx, semA).wait()
    plsc.subcore_barrier()

    def gath(c, buf, sem):
        pltpu.async_copy(x_hbm.at[sidx.at[pl.ds(c * EC, EC)]], buf, sem)

    def gwait(buf, sem):
        pltpu.make_async_copy(x_hbm.at[pl.ds(0, EC)], buf, sem).wait()

    def scat(c, buf):
        pltpu.sync_copy(buf, agg_sh.at[didx.at[pl.ds(c * EC, EC)]], add=True)

    gath(0, rowsA, semA)

    @pl.loop(0, (NCPW - 1) // 2)
    def _(j2):
        c = 2 * j2
        gath(c + 1, rowsB, semB)
        gwait(rowsA, semA)
        scat(c, rowsA)
        gath(c + 2, rowsA, semA)
        gwait(rowsB, semB)
        scat(c + 1, rowsB)

    gwait(rowsA, semA)
    scat(NCPW - 1, rowsA)

    plsc.subcore_barrier()

    obufs = (rowsA, rowsB)
    osems = (semA, semB)
    for k in range(OB // EC):
        o = sid * OB + k * EC
        b, sm = obufs[k % 2], osems[k % 2]
        if k >= 2:
            op = sid * OB + (k - 2) * EC
            pltpu.make_async_copy(b, out_hbm.at[cid, pl.ds(op, EC)], sm).wait()
        pltpu.sync_copy(agg_sh.at[pl.ds(o, EC)], b)
        pltpu.async_copy(b, out_hbm.at[cid, pl.ds(o, EC)], sm)
    for k in (OB // EC - 2, OB // EC - 1):
        o = sid * OB + k * EC
        pltpu.make_async_copy(obufs[k % 2], out_hbm.at[cid, pl.ds(o, EC)],
                              osems[k % 2]).wait()


@functools.partial(
    pl.kernel,
    out_type=(
        jax.ShapeDtypeStruct((NC, GP, D), jnp.float32),
        jax.ShapeDtypeStruct((NW, GM, D), jnp.float32),
    ),
    mesh=_MESH,
    scratch_types=[
        pltpu.VMEM((RPW,), jnp.int32),
        pltpu.VMEM((RPW, D), jnp.float32),
        pltpu.VMEM((GM, D), jnp.float32),
        pltpu.VMEM_SHARED((GP, D), jnp.float32),
        pltpu.SemaphoreType.DMA,
    ],
)
def _sc_pool(x_hbm, batch_hbm, sum_hbm, max_hbm, bidx, rows, lmax, sum_sh, sem):
    cid = lax.axis_index("c")
    sid = lax.axis_index("s")
    wid = sid * NC + cid
    base = wid * RPW

    pltpu.async_copy(batch_hbm.at[pl.ds(base, RPW)], bidx, sem)
    pltpu.async_copy(x_hbm.at[pl.ds(base, RPW)], rows, sem)
    _zero_rows(lmax, GM)
    pltpu.sync_copy(lmax.at[pl.ds(0, GP // NS)],
                    sum_sh.at[pl.ds(sid * (GP // NS), GP // NS)])
    pltpu.make_async_copy(batch_hbm.at[pl.ds(base, RPW)], bidx, sem).wait()
    pltpu.make_async_copy(x_hbm.at[pl.ds(base, RPW)], rows, sem).wait()
    plsc.subcore_barrier()

    for j in range(RCH):
        pltpu.async_copy(rows.at[pl.ds(j * EC, EC)],
                         sum_sh.at[bidx.at[pl.ds(j * EC, EC)]], sem, add=True)

    @pl.loop(0, RPW // 16)
    def _(q):
        bvec = bidx[pl.ds(q * 16, 16)]
        for r in range(16):
            g = bvec[r]
            for c in range(D // 16):
                sl = pl.ds(c * 16, 16)
                lmax[g, sl] = jnp.maximum(lmax[g, sl],
                                          rows[q * 16 + r, sl])

    pltpu.sync_copy(lmax, max_hbm.at[wid])

    for j in range(RCH):
        pltpu.make_async_copy(rows.at[pl.ds(j * EC, EC)],
                              sum_sh.at[bidx.at[pl.ds(j * EC, EC)]], sem).wait()
    plsc.subcore_barrier()
    o = sid * (GP // NS)
    pltpu.sync_copy(sum_sh.at[pl.ds(o, GP // NS)], lmax.at[pl.ds(0, GP // NS)])
    pltpu.sync_copy(lmax.at[pl.ds(0, GP // NS)],
                    sum_hbm.at[cid, pl.ds(o, GP // NS)])


_RB = 1024


def _tc_layer_body(aggp_ref, x_ref, cntp_ref, wl_ref, wr_ref, b_ref, o_ref):
    i = pl.program_id(0)
    cnt = cntp_ref[0, pl.ds(i * _RB, _RB)] + cntp_ref[1, pl.ds(i * _RB, _RB)]
    inv = 1.0 / jnp.maximum(cnt, 1.0)
    agg = (aggp_ref[0] + aggp_ref[1]) * inv.reshape(_RB, 1)
    acc = lax.dot_general(agg, wl_ref[...], (((1,), (1,)), ((), ())),
                          preferred_element_type=jnp.float32)
    acc += lax.dot_general(x_ref[...], wr_ref[...], (((1,), (1,)), ((), ())),
                           preferred_element_type=jnp.float32)
    o_ref[...] = jnp.maximum(acc + b_ref[...], 0.0)


def _tc_layer(aggp, x, cntp, wl, wr, b2d):
    return pl.pallas_call(
        _tc_layer_body,
        grid=(NP // _RB,),
        in_specs=[
            pl.BlockSpec((NC, _RB, D), lambda i: (0, i, 0)),
            pl.BlockSpec((_RB, D), lambda i: (i, 0)),
            pl.BlockSpec((NC, NP), lambda i: (0, 0)),
            pl.BlockSpec((D, D), lambda i: (0, 0)),
            pl.BlockSpec((D, D), lambda i: (0, 0)),
            pl.BlockSpec((1, D), lambda i: (0, 0)),
        ],
        out_specs=pl.BlockSpec((_RB, D), lambda i: (i, 0)),
        out_shape=jax.ShapeDtypeStruct((NP, D), jnp.float32),
    )(aggp, x, cntp, wl, wr, b2d)


def _tc_final_body(sump_ref, maxp_ref, cntgp_ref, a2p_ref,
                   lin1w_ref, lin1b_ref,
                   wl0_ref, wr0_ref, wd0_ref, b0_ref,
                   wl1_ref, wr1_ref, wd1_ref, b1_ref, o_ref):
    gm = maxp_ref[0, 0:G, :]
    for k in range(1, NW):
        gm = jnp.maximum(gm, maxp_ref[k, 0:G, :])
    ga = sump_ref[0, 0:G, :] + sump_ref[1, 0:G, :]
    cg = cntgp_ref[0, pl.ds(0, G)] + cntgp_ref[1, pl.ds(0, G)]
    ga = ga * (1.0 / jnp.maximum(cg, 1.0)).reshape(G, 1)
    xcat = jnp.concatenate([gm, ga], axis=1)
    xdrug = lax.dot_general(xcat, lin1w_ref[...], (((1,), (1,)), ((), ())),
                            preferred_element_type=jnp.float32)
    xdrug = jnp.maximum(xdrug + lin1b_ref[...], 0.0)

    a2 = a2p_ref[0] + a2p_ref[1]
    cnt2 = jnp.sum(a2, axis=1, keepdims=True)
    an = a2 / jnp.maximum(cnt2, 1.0)

    x2 = xdrug
    for (wl, wr, wd, b) in ((wl0_ref, wr0_ref, wd0_ref, b0_ref),
                            (wl1_ref, wr1_ref, wd1_ref, b1_ref)):
        agg2 = lax.dot_general(an, x2, (((1,), (0,)), ((), ())),
                               preferred_element_type=jnp.float32)
        acc = lax.dot_general(agg2, wl[...], (((1,), (1,)), ((), ())),
                              preferred_element_type=jnp.float32)
        acc += lax.dot_general(x2, wr[...], (((1,), (1,)), ((), ())),
                               preferred_element_type=jnp.float32)
        acc += lax.dot_general(xdrug, wd[...], (((1,), (1,)), ((), ())),
                               preferred_element_type=jnp.float32)
        x2 = jnp.maximum(acc + b[...], 0.0)
    o_ref[...] = x2


def _tc_final(sump, maxp, cntgp, a2p, lin1w, lin1b2d,
              wl0, wr0, wd0, b02d, wl1, wr1, wd1, b12d):
    return pl.pallas_call(
        _tc_final_body,
        out_shape=jax.ShapeDtypeStruct((G, D), jnp.float32),
    )(sump, maxp, cntgp, a2p, lin1w, lin1b2d,
      wl0, wr0, wd0, b02d, wl1, wr1, wd1, b12d)


def kernel(emb1, emb2, Wl1_0, Wr1_0, b1_0, Wl1_1, Wr1_1, b1_1, lin1_W, lin1_b,
           Wl2_0, Wr2_0, Wd2_0, b2_0, Wl2_1, Wr2_1, Wd2_1, b2_1,
           x1, edge_index1, batch1, x2_idx, edge_index2):
    x1p = jnp.concatenate([x1, jnp.full((NP - N1,), N1, jnp.int32)])
    batchp = jnp.concatenate([batch1, jnp.full((NP - N1,), G, jnp.int32)])
    src1 = edge_index1[0]
    dst1 = edge_index1[1]
    src2, dst2 = edge_index2[0], edge_index2[1]

    xA, cntp, cntgp, a2p_flat = _sc_prep(emb1, x1p, dst1, batchp, src2, dst2)
    a2p = a2p_flat.reshape(NC, G, G)

    aggp = _sc_agg(xA, src1, dst1)
    xA = _tc_layer(aggp, xA, cntp, Wl1_0, Wr1_0, b1_0.reshape(1, D))
    aggp = _sc_agg(xA, src1, dst1)
    xA = _tc_layer(aggp, xA, cntp, Wl1_1, Wr1_1, b1_1.reshape(1, D))

    sump, maxp = _sc_pool(xA, batchp)

    return _tc_final(sump, maxp, cntgp, a2p, lin1_W, lin1_b.reshape(1, D),
                     Wl2_0, Wr2_0, Wd2_0, b2_0.reshape(1, D),
                     Wl2_1, Wr2_1, Wd2_1, b2_1.reshape(1, D))


# speedup vs baseline: 11.1653x; 1.0005x over previous
import functools

import jax
import jax.numpy as jnp
from jax import lax
from jax.experimental import pallas as pl
from jax.experimental.pallas import tpu as pltpu
from jax.experimental.pallas import tpu_sc as plsc

D = 128
N1 = 10000
NP = 10240
E1 = 320000
G = 256
GP = 512
GM = 272
E2 = 4096

NC = 2
NS = 16
NW = NC * NS

EC = 80
ECH = E1 // EC
NCPW = ECH // NW
EPW = E1 // NW

RPW = NP // NW
RCH = RPW // EC
OB = NP // NS

_MESH = plsc.VectorSubcoreMesh(core_axis_name="c", subcore_axis_name="s",
                               num_cores=NC, num_subcores=NS)


def _wid():
    return lax.axis_index("s") * NC + lax.axis_index("c")


def _zero_rows(buf, nrows):
    z = jnp.zeros((16,), jnp.float32)

    @pl.loop(0, nrows)
    def _(r):
        for c in range(D // 16):
            buf[r, pl.ds(c * 16, 16)] = z


def _fill_1d(buf, n, val):
    v = jnp.full((16,), val, jnp.float32)

    @pl.loop(0, n // 16)
    def _(k):
        buf[pl.ds(k * 16, 16)] = v


@functools.partial(
    pl.kernel,
    out_type=(
        jax.ShapeDtypeStruct((NP, D), jnp.float32),
        jax.ShapeDtypeStruct((NC, NP), jnp.float32),
        jax.ShapeDtypeStruct((NC, GP), jnp.float32),
        jax.ShapeDtypeStruct((NC, G * G), jnp.float32),
    ),
    mesh=_MESH,
    scratch_types=[
        pltpu.VMEM((RPW,), jnp.int32),
        pltpu.VMEM((EPW,), jnp.int32),
        pltpu.VMEM((RPW,), jnp.int32),
        pltpu.VMEM((64,), jnp.int32),
        pltpu.VMEM((64,), jnp.int32),
        pltpu.VMEM((64,), jnp.int32),
        pltpu.VMEM((128,), jnp.float32),
        pltpu.VMEM((640,), jnp.float32),
        pltpu.VMEM((EC, D), jnp.float32),
        pltpu.VMEM((EC, D), jnp.float32),
        pltpu.VMEM_SHARED((NP,), jnp.float32),
        pltpu.VMEM_SHARED((GP,), jnp.float32),
        pltpu.VMEM_SHARED((G * G,), jnp.float32),
        pltpu.SemaphoreType.DMA,
        pltpu.SemaphoreType.DMA,
        pltpu.SemaphoreType.DMA,
    ],
)
def _sc_prep(emb_hbm, x1_hbm, dst1_hbm, batch_hbm, src2_hbm, dst2_hbm,
             xa_hbm, cnt_hbm, cntg_hbm, a2_hbm,
             gidx, didx, bidx, s2, d2, f2, ones_v, zb,
             rowsA, rowsB, cnt_sh, cntg_sh, a2_sh, semA, semB, ssem):
    cid = lax.axis_index("c")
    sid = lax.axis_index("s")
    wid = sid * NC + cid

    _fill_1d(ones_v, 128, 1.0)
    _fill_1d(zb, 640, 0.0)

    pltpu.sync_copy(zb, cnt_sh.at[pl.ds(sid * OB, OB)])
    pltpu.sync_copy(zb.at[pl.ds(0, GP // NS)],
                    cntg_sh.at[pl.ds(sid * (GP // NS), GP // NS)])
    a2pt = G * G // NS

    @pl.loop(0, 6)
    def _(k):
        pltpu.sync_copy(zb, a2_sh.at[pl.ds(sid * a2pt + k * 640, 640)])

    pltpu.sync_copy(zb.at[pl.ds(0, 256)],
                    a2_sh.at[pl.ds(sid * a2pt + 3840, 256)])

    rbase = wid * RPW
    pltpu.sync_copy(x1_hbm.at[pl.ds(rbase, RPW)], gidx)
    bufs = (rowsA, rowsB)
    sems = (semA, semB)
    pltpu.async_copy(emb_hbm.at[gidx.at[pl.ds(0, EC)]], rowsA, semA)
    for c in range(RCH):
        if c + 1 < RCH:
            pltpu.async_copy(emb_hbm.at[gidx.at[pl.ds((c + 1) * EC, EC)]],
                             bufs[(c + 1) % 2], sems[(c + 1) % 2])
        pltpu.make_async_copy(emb_hbm.at[pl.ds(0, EC)], bufs[c % 2],
                              sems[c % 2]).wait()
        pltpu.sync_copy(bufs[c % 2], xa_hbm.at[pl.ds(rbase + c * EC, EC)])

    plsc.subcore_barrier()

    pltpu.sync_copy(dst1_hbm.at[pl.ds(wid * EPW, EPW)], didx)

    @pl.loop(0, 5)
    def _(blk):
        for j in range(25):
            pltpu.async_copy(
                ones_v.at[pl.ds(0, EC)],
                cnt_sh.at[didx.at[pl.ds((blk * 25 + j) * EC, EC)]],
                ssem, add=True)
        for j in range(25):
            pltpu.make_async_copy(
                ones_v.at[pl.ds(0, EC)],
                cnt_sh.at[didx.at[pl.ds((blk * 25 + j) * EC, EC)]],
                ssem).wait()

    pltpu.sync_copy(batch_hbm.at[pl.ds(wid * RPW, RPW)], bidx)
    for j in range(RCH):
        pltpu.sync_copy(ones_v.at[pl.ds(0, EC)],
                        cntg_sh.at[bidx.at[pl.ds(j * EC, EC)]], add=True)

    e2base = wid * (E2 // NW)
    for j in range(2):
        b = e2base + j * 64
        pltpu.sync_copy(src2_hbm.at[pl.ds(b, 64)], s2)
        pltpu.sync_copy(dst2_hbm.at[pl.ds(b, 64)], d2)
        for c in range(4):
            f2[pl.ds(c * 16, 16)] = (
                d2[pl.ds(c * 16, 16)] * G + s2[pl.ds(c * 16, 16)])
        pltpu.sync_copy(ones_v.at[pl.ds(0, 64)], a2_sh.at[f2], add=True)

    plsc.subcore_barrier()

    pltpu.sync_copy(cnt_sh.at[pl.ds(sid * OB, OB)], zb)
    pltpu.sync_copy(zb, cnt_hbm.at[cid, pl.ds(sid * OB, OB)])

    og = sid * (GP // NS)
    pltpu.sync_copy(cntg_sh.at[pl.ds(og, GP // NS)], zb.at[pl.ds(0, GP // NS)])
    pltpu.sync_copy(zb.at[pl.ds(0, GP // NS)],
                    cntg_hbm.at[cid, pl.ds(og, GP // NS)])

    @pl.loop(0, 6)
    def _(k):
        o = sid * a2pt + k * 640
        pltpu.sync_copy(a2_sh.at[pl.ds(o, 640)], zb)
        pltpu.sync_copy(zb, a2_hbm.at[cid, pl.ds(o, 640)])

    o = sid * a2pt + 3840
    pltpu.sync_copy(a2_sh.at[pl.ds(o, 256)], zb.at[pl.ds(0, 256)])
    pltpu.sync_copy(zb.at[pl.ds(0, 256)], a2_hbm.at[cid, pl.ds(o, 256)])


@functools.partial(
    pl.kernel,
    out_type=jax.ShapeDtypeStruct((NC, NP, D), jnp.float32),
    mesh=_MESH,
    scratch_types=[
        pltpu.VMEM((EPW,), jnp.int32),
        pltpu.VMEM((EPW,), jnp.int32),
        pltpu.VMEM((EC, D), jnp.float32),
        pltpu.VMEM((EC, D), jnp.float32),
        pltpu.VMEM_SHARED((NP, D), jnp.float32),
        pltpu.SemaphoreType.DMA,
        pltpu.SemaphoreType.DMA,
    ],
)
def _sc_agg(x_hbm, src_hbm, dst_hbm, out_hbm,
            sidx, didx, rowsA, rowsB, agg_sh, semA, semB):
    cid = lax.axis_index("c")
    sid = lax.axis_index("s")
    wid = sid * NC + cid

    pltpu.async_copy(src_hbm.at[pl.ds(wid * EPW, EPW)], sidx, semA)
    pltpu.async_copy(dst_hbm.at[pl.ds(wid * EPW, EPW)], didx, semA)

    _zero_rows(rowsA, EC)

    @pl.loop(0, OB // EC)
    def _(k):
        pltpu.sync_copy(rowsA, agg_sh.at[pl.ds(sid * OB + k * EC, EC)])

    pltpu.make_async_copy(src_hbm.at[pl.ds(wid * EPW, EPW)], sidx, semA).wait()
    pltpu.make_async_copy(dst_hbm.at[pl.ds(wid * EPW, EPW)], didx, semA).wait()
    plsc.subcore_barrier()

    def gath(c, buf, sem):
        pltpu.async_copy(x_hbm.at[sidx.at[pl.ds(c * EC, EC)]], buf, sem)

    def gwait(buf, sem):
        pltpu.make_async_copy(x_hbm.at[pl.ds(0, EC)], buf, sem).wait()

    def scat(c, buf):
        pltpu.sync_copy(buf, agg_sh.at[didx.at[pl.ds(c * EC, EC)]], add=True)

    gath(0, rowsA, semA)

    @pl.loop(0, (NCPW - 1) // 2)
    def _(j2):
        c = 2 * j2
        gath(c + 1, rowsB, semB)
        gwait(rowsA, semA)
        scat(c, rowsA)
        gath(c + 2, rowsA, semA)
        gwait(rowsB, semB)
        scat(c + 1, rowsB)

    gwait(rowsA, semA)
    scat(NCPW - 1, rowsA)

    plsc.subcore_barrier()

    obufs = (rowsA, rowsB)
    osems = (semA, semB)
    for k in range(OB // EC):
        o = sid * OB + k * EC
        b, sm = obufs[k % 2], osems[k % 2]
        if k >= 2:
            op = sid * OB + (k - 2) * EC
            pltpu.make_async_copy(b, out_hbm.at[cid, pl.ds(op, EC)], sm).wait()
        pltpu.sync_copy(agg_sh.at[pl.ds(o, EC)], b)
        pltpu.async_copy(b, out_hbm.at[cid, pl.ds(o, EC)], sm)
    for k in (OB // EC - 2, OB // EC - 1):
        o = sid * OB + k * EC
        pltpu.make_async_copy(obufs[k % 2], out_hbm.at[cid, pl.ds(o, EC)],
                              osems[k % 2]).wait()


@functools.partial(
    pl.kernel,
    out_type=(
        jax.ShapeDtypeStruct((NC, GP, D), jnp.float32),
        jax.ShapeDtypeStruct((NW, GM, D), jnp.float32),
    ),
    mesh=_MESH,
    scratch_types=[
        pltpu.VMEM((RPW,), jnp.int32),
        pltpu.VMEM((RPW, D), jnp.float32),
        pltpu.VMEM((GM, D), jnp.float32),
        pltpu.VMEM_SHARED((GP, D), jnp.float32),
        pltpu.SemaphoreType.DMA,
    ],
)
def _sc_pool(x_hbm, batch_hbm, sum_hbm, max_hbm, bidx, rows, lmax, sum_sh, sem):
    cid = lax.axis_index("c")
    sid = lax.axis_index("s")
    wid = sid * NC + cid
    base = wid * RPW

    pltpu.async_copy(batch_hbm.at[pl.ds(base, RPW)], bidx, sem)
    pltpu.async_copy(x_hbm.at[pl.ds(base, RPW)], rows, sem)
    _zero_rows(lmax, GM)
    pltpu.sync_copy(lmax.at[pl.ds(0, GP // NS)],
                    sum_sh.at[pl.ds(sid * (GP // NS), GP // NS)])
    pltpu.make_async_copy(batch_hbm.at[pl.ds(base, RPW)], bidx, sem).wait()
    pltpu.make_async_copy(x_hbm.at[pl.ds(base, RPW)], rows, sem).wait()
    plsc.subcore_barrier()

    for j in range(RCH):
        pltpu.async_copy(rows.at[pl.ds(j * EC, EC)],
                         sum_sh.at[bidx.at[pl.ds(j * EC, EC)]], sem, add=True)

    @pl.loop(0, RPW // 16)
    def _(q):
        bvec = bidx[pl.ds(q * 16, 16)]
        for r in range(16):
            g = bvec[r]
            for c in range(D // 16):
                sl = pl.ds(c * 16, 16)
                lmax[g, sl] = jnp.maximum(lmax[g, sl],
                                          rows[q * 16 + r, sl])

    pltpu.sync_copy(lmax, max_hbm.at[wid])

    for j in range(RCH):
        pltpu.make_async_copy(rows.at[pl.ds(j * EC, EC)],
                              sum_sh.at[bidx.at[pl.ds(j * EC, EC)]], sem).wait()
    plsc.subcore_barrier()
    o = sid * (GP // NS)
    pltpu.sync_copy(sum_sh.at[pl.ds(o, GP // NS)], lmax.at[pl.ds(0, GP // NS)])
    pltpu.sync_copy(lmax.at[pl.ds(0, GP // NS)],
                    sum_hbm.at[cid, pl.ds(o, GP // NS)])


_RB = 1024


def _tc_layer_body(aggp_ref, x_ref, cntp_ref, wl_ref, wr_ref, b_ref, o_ref):
    i = pl.program_id(0)
    cnt = cntp_ref[0, pl.ds(i * _RB, _RB)] + cntp_ref[1, pl.ds(i * _RB, _RB)]
    inv = 1.0 / jnp.maximum(cnt, 1.0)
    agg = (aggp_ref[0] + aggp_ref[1]) * inv.reshape(_RB, 1)
    acc = lax.dot_general(agg, wl_ref[...], (((1,), (1,)), ((), ())),
                          preferred_element_type=jnp.float32)
    acc += lax.dot_general(x_ref[...], wr_ref[...], (((1,), (1,)), ((), ())),
                           preferred_element_type=jnp.float32)
    o_ref[...] = jnp.maximum(acc + b_ref[...], 0.0)


def _tc_layer(aggp, x, cntp, wl, wr, b2d):
    return pl.pallas_call(
        _tc_layer_body,
        grid=(NP // _RB,),
        in_specs=[
            pl.BlockSpec((NC, _RB, D), lambda i: (0, i, 0)),
            pl.BlockSpec((_RB, D), lambda i: (i, 0)),
            pl.BlockSpec((NC, NP), lambda i: (0, 0)),
            pl.BlockSpec((D, D), lambda i: (0, 0)),
            pl.BlockSpec((D, D), lambda i: (0, 0)),
            pl.BlockSpec((1, D), lambda i: (0, 0)),
        ],
        out_specs=pl.BlockSpec((_RB, D), lambda i: (i, 0)),
        out_shape=jax.ShapeDtypeStruct((NP, D), jnp.float32),
    )(aggp, x, cntp, wl, wr, b2d)


def _tc_final_body(sump_ref, maxp_ref, cntgp_ref, a2p_ref,
                   lin1w_ref, lin1b_ref,
                   wl0_ref, wr0_ref, wd0_ref, b0_ref,
                   wl1_ref, wr1_ref, wd1_ref, b1_ref, o_ref):
    gm = maxp_ref[0, 0:G, :]
    for k in range(1, NW):
        gm = jnp.maximum(gm, maxp_ref[k, 0:G, :])
    ga = sump_ref[0, 0:G, :] + sump_ref[1, 0:G, :]
    cg = cntgp_ref[0, pl.ds(0, G)] + cntgp_ref[1, pl.ds(0, G)]
    ga = ga * (1.0 / jnp.maximum(cg, 1.0)).reshape(G, 1)
    xcat = jnp.concatenate([gm, ga], axis=1)
    xdrug = lax.dot_general(xcat, lin1w_ref[...], (((1,), (1,)), ((), ())),
                            preferred_element_type=jnp.float32)
    xdrug = jnp.maximum(xdrug + lin1b_ref[...], 0.0)

    a2 = a2p_ref[0] + a2p_ref[1]
    cnt2 = jnp.sum(a2, axis=1, keepdims=True)
    an = a2 / jnp.maximum(cnt2, 1.0)

    x2 = xdrug
    for (wl, wr, wd, b) in ((wl0_ref, wr0_ref, wd0_ref, b0_ref),
                            (wl1_ref, wr1_ref, wd1_ref, b1_ref)):
        agg2 = lax.dot_general(an, x2, (((1,), (0,)), ((), ())),
                               preferred_element_type=jnp.float32)
        acc = lax.dot_general(agg2, wl[...], (((1,), (1,)), ((), ())),
                              preferred_element_type=jnp.float32)
        acc += lax.dot_general(x2, wr[...], (((1,), (1,)), ((), ())),
                               preferred_element_type=jnp.float32)
        acc += lax.dot_general(xdrug, wd[...], (((1,), (1,)), ((), ())),
                               preferred_element_type=jnp.float32)
        x2 = jnp.maximum(acc + b[...], 0.0)
    o_ref[...] = x2


def _tc_final(sump, maxp, cntgp, a2p, lin1w, lin1b2d,
              wl0, wr0, wd0, b02d, wl1, wr1, wd1, b12d):
    return pl.pallas_call(
        _tc_final_body,
        out_shape=jax.ShapeDtypeStruct((G, D), jnp.float32),
    )(sump, maxp, cntgp, a2p, lin1w, lin1b2d,
      wl0, wr0, wd0, b02d, wl1, wr1, wd1, b12d)


def kernel(emb1, emb2, Wl1_0, Wr1_0, b1_0, Wl1_1, Wr1_1, b1_1, lin1_W, lin1_b,
           Wl2_0, Wr2_0, Wd2_0, b2_0, Wl2_1, Wr2_1, Wd2_1, b2_1,
           x1, edge_index1, batch1, x2_idx, edge_index2):
    x1p = jnp.concatenate([x1, jnp.full((NP - N1,), N1, jnp.int32)])
    batchp = jnp.concatenate([batch1, jnp.full((NP - N1,), G, jnp.int32)])
    src1 = edge_index1[0]
    dst1 = edge_index1[1]
    src2, dst2 = edge_index2[0], edge_index2[1]

    xA, cntp, cntgp, a2p_flat = _sc_prep(emb1, x1p, dst1, batchp, src2, dst2)
    a2p = a2p_flat.reshape(NC, G, G)

    aggp = _sc_agg(xA, src1, dst1)
    xA = _tc_layer(aggp, xA, cntp, Wl1_0, Wr1_0, b1_0.reshape(1, D))
    aggp = _sc_agg(xA, src1, dst1)
    xA = _tc_layer(aggp, xA, cntp, Wl1_1, Wr1_1, b1_1.reshape(1, D))

    sump, maxp = _sc_pool(xA, batchp)

    return _tc_final(sump, maxp, cntgp, a2p, lin1_W, lin1_b.reshape(1, D),
                     Wl2_0, Wr2_0, Wd2_0, b2_0.reshape(1, D),
                     Wl2_1, Wr2_1, Wd2_1, b2_1.reshape(1, D))
